# Initial kernel scaffold; baseline (speedup 1.0000x reference)
import jax
import jax.numpy as jnp
from jax.experimental import pallas as pl


def kernel(x, edge_index, edge_attr, W_in1, b_in1, ga_in1, be_in1, W_in2, b_in2, ga_in2, be_in2, g0, mu0, sigma0, root0, bias0, g1, mu1, sigma1, root1, bias1, g2, mu2, sigma2, root2, bias2, W_o1, b_o1, W_o2, b_o2):
    raise NotImplementedError("write your pallas kernel here")


# speedup vs baseline: 3.2781x; 3.2781x over previous
import functools

import jax
import jax.numpy as jnp
from jax import lax
from jax.experimental import pallas as pl
from jax.experimental.pallas import tpu as pltpu
from jax.experimental.pallas import tpu_sc as plsc

N = 10000
E = 320000
DF = 128
DH = 16
ED = 4
K = 6
YW = K * DH

NB = 400
GRID = N // NB
EB = 2560
NWORK = 32
EW = E // NWORK
BE = 80
NSTEP = EW // BE
STRIPE = N // 16


def _lin_in_body(x_ref, w1_ref, b1_ref, w2_ref, b2_ref, g_ref, rt_ref, bi_ref,
                 y_ref, r_ref):
    h = jnp.dot(x_ref[...], w1_ref[...], preferred_element_type=jnp.float32)
    h = jax.nn.relu(h + b1_ref[...])
    h = jnp.dot(h, w2_ref[...], preferred_element_type=jnp.float32)
    h = jax.nn.relu(h + b2_ref[...])
    y_ref[...] = jnp.dot(h, g_ref[...], preferred_element_type=jnp.float32)
    r_ref[...] = jnp.dot(h, rt_ref[...], preferred_element_type=jnp.float32) + bi_ref[...]


def _lin_in(x, W1f, b1f, W2f, b2f, g0, root0, bias0):
    full = lambda shape: pl.BlockSpec(shape, lambda i: (0,) * len(shape))
    return pl.pallas_call(
        _lin_in_body,
        grid=(GRID,),
        in_specs=[
            pl.BlockSpec((NB, DF), lambda i: (i, 0)),
            full((DF, DF)), full((1, DF)), full((DF, DH)), full((1, DH)),
            full((DH, YW)), full((DH, DH)), full((1, DH)),
        ],
        out_specs=[
            pl.BlockSpec((NB, YW), lambda i: (i, 0)),
            pl.BlockSpec((NB, DH), lambda i: (i, 0)),
        ],
        out_shape=[
            jax.ShapeDtypeStruct((N, YW), jnp.float32),
            jax.ShapeDtypeStruct((N, DH), jnp.float32),
        ],
    )(x, W1f, b1f, W2f, b2f, g0, root0, bias0)


def _gauss_body(ea_ref, c0_ref, c1_ref, c2_ref, o0_ref, o1_ref, o2_ref):
    ea = ea_ref[...]
    ea2 = ea * ea
    for c_ref, o_ref in ((c0_ref, o0_ref), (c1_ref, o1_ref), (c2_ref, o2_ref)):
        cc = c_ref[...]
        acc = cc[:, 8:9]
        for d in range(ED):
            acc = acc + cc[:, d:d + 1] * ea2[d:d + 1, :] + cc[:, ED + d:ED + d + 1] * ea[d:d + 1, :]
        o_ref[...] = jnp.exp(acc)


def _gauss(eaT, c0, c1, c2):
    full = lambda shape: pl.BlockSpec(shape, lambda i: (0,) * len(shape))
    ospec = pl.BlockSpec((8, EB), lambda i: (0, i))
    oshape = jax.ShapeDtypeStruct((8, E), jnp.float32)
    return pl.pallas_call(
        _gauss_body,
        grid=(E // EB,),
        in_specs=[pl.BlockSpec((4, EB), lambda i: (0, i)),
                  full((8, 16)), full((8, 16)), full((8, 16))],
        out_specs=[ospec, ospec, ospec],
        out_shape=[oshape, oshape, oshape],
    )(eaT, c0, c1, c2)


def _sc_body(with_cnt, y_hbm, src_hbm, dst_hbm, g_hbm, zeros_hbm,
             acc_out, cnt_out,
             src_v, dst_v, y_blk, g_blk, msg_v, ones_v, accum, cnt_accum, sem):
    c = lax.axis_index("c")
    s = lax.axis_index("s")
    w = s * 2 + c
    base_e = w * EW

    pltpu.sync_copy(zeros_hbm.at[pl.ds(s * STRIPE, STRIPE)],
                    accum.at[pl.ds(s * STRIPE, STRIPE)])
    if with_cnt:
        pltpu.sync_copy(zeros_hbm.at[pl.ds(s * STRIPE, STRIPE)],
                        cnt_accum.at[pl.ds(s * STRIPE, STRIPE)])

        def fill_ones(j, carry):
            ones_v[j, :] = jnp.full((16,), 1.0, jnp.float32)
            return carry
        lax.fori_loop(0, BE, fill_ones, 0)
    plsc.subcore_barrier()

    def block_body(j, carry):
        eb = base_e + j * BE
        pltpu.sync_copy(src_hbm.at[pl.ds(eb, BE)], src_v)
        pltpu.sync_copy(dst_hbm.at[pl.ds(eb, BE)], dst_v)
        pltpu.sync_copy(g_hbm.at[:, pl.ds(eb, BE)], g_blk)
        pltpu.async_copy(y_hbm.at[src_v], y_blk, sem).wait()

        def group_body(gi, carry2):
            e0 = gi * 16
            gv = [g_blk[k, pl.ds(e0, 16)] for k in range(K)]
            for jj in range(16):
                e = e0 + jj
                acc = gv[0][jj] * y_blk[e, pl.ds(0, 16)]
                for k in range(1, K):
                    acc = acc + gv[k][jj] * y_blk[e, pl.ds(k * 16, 16)]
                msg_v[e, :] = acc
            return carry2
        lax.fori_loop(0, BE // 16, group_body, 0)

        pltpu.sync_copy(msg_v, accum.at[dst_v], add=True)
        if with_cnt:
            pltpu.sync_copy(ones_v, cnt_accum.at[dst_v], add=True)
        return carry
    lax.fori_loop(0, NSTEP, block_body, 0)

    plsc.subcore_barrier()
    pltpu.sync_copy(accum.at[pl.ds(s * STRIPE, STRIPE)],
                    acc_out.at[c, pl.ds(s * STRIPE, STRIPE)])
    if with_cnt:
        pltpu.sync_copy(cnt_accum.at[pl.ds(s * STRIPE, STRIPE)],
                        cnt_out.at[c, pl.ds(s * STRIPE, STRIPE)])


def _make_sc(with_cnt):
    mesh = plsc.VectorSubcoreMesh(core_axis_name="c", subcore_axis_name="s",
                                  num_cores=2, num_subcores=16)
    out_type = [jax.ShapeDtypeStruct((2, N, DH), jnp.float32),
                jax.ShapeDtypeStruct((2, N, DH), jnp.float32)]
    scratch = [
        pltpu.VMEM((BE,), jnp.int32),
        pltpu.VMEM((BE,), jnp.int32),
        pltpu.VMEM((BE, YW), jnp.float32),
        pltpu.VMEM((8, BE), jnp.float32),
        pltpu.VMEM((BE, DH), jnp.float32),
        pltpu.VMEM((BE, DH), jnp.float32),
        pltpu.VMEM_SHARED((N, DH), jnp.float32),
        pltpu.VMEM_SHARED((N, DH), jnp.float32),
        pltpu.SemaphoreType.DMA,
    ]
    return pl.kernel(
        functools.partial(_sc_body, with_cnt),
        out_type=out_type,
        mesh=mesh,
        scratch_types=scratch,
        compiler_params=pltpu.CompilerParams(use_tc_tiling_on_sc=False),
    )


@functools.lru_cache(maxsize=None)
def _sc_pass_fn(with_cnt):
    return _make_sc(with_cnt)


def _combine_body(acc_ref, cnt_ref, r_ref, g_ref, rt_ref, bi_ref, y_ref, rn_ref):
    s = acc_ref[0] + acc_ref[1]
    c = cnt_ref[0] + cnt_ref[1]
    aggr = s / jnp.maximum(c, 1.0)
    h = jax.nn.relu(aggr + r_ref[...])
    y_ref[...] = jnp.dot(h, g_ref[...], preferred_element_type=jnp.float32)
    rn_ref[...] = jnp.dot(h, rt_ref[...], preferred_element_type=jnp.float32) + bi_ref[...]


def _combine(acc, cnt, r, g_next, root_next, bias_next):
    full = lambda shape: pl.BlockSpec(shape, lambda i: (0,) * len(shape))
    return pl.pallas_call(
        _combine_body,
        grid=(GRID,),
        in_specs=[
            pl.BlockSpec((2, NB, DH), lambda i: (0, i, 0)),
            pl.BlockSpec((2, NB, DH), lambda i: (0, i, 0)),
            pl.BlockSpec((NB, DH), lambda i: (i, 0)),
            full((DH, YW)), full((DH, DH)), full((1, DH)),
        ],
        out_specs=[
            pl.BlockSpec((NB, YW), lambda i: (i, 0)),
            pl.BlockSpec((NB, DH), lambda i: (i, 0)),
        ],
        out_shape=[
            jax.ShapeDtypeStruct((N, YW), jnp.float32),
            jax.ShapeDtypeStruct((N, DH), jnp.float32),
        ],
    )(acc, cnt, r, g_next, root_next, bias_next)


def _combine_out_body(acc_ref, cnt_ref, r_ref, w1_ref, b1_ref, w2_ref, b2_ref, o_ref):
    s = acc_ref[0] + acc_ref[1]
    c = cnt_ref[0] + cnt_ref[1]
    aggr = s / jnp.maximum(c, 1.0)
    h = jax.nn.relu(aggr + r_ref[...])
    t = jax.nn.relu(jnp.dot(h, w1_ref[...], preferred_element_type=jnp.float32) + b1_ref[...])
    o_ref[...] = jnp.dot(t, w2_ref[...], preferred_element_type=jnp.float32) + b2_ref[...]


def _combine_out(acc, cnt, r, W_o1, b_o1, W_o2, b_o2):
    full = lambda shape: pl.BlockSpec(shape, lambda i: (0,) * len(shape))
    return pl.pallas_call(
        _combine_out_body,
        grid=(GRID,),
        in_specs=[
            pl.BlockSpec((2, NB, DH), lambda i: (0, i, 0)),
            pl.BlockSpec((2, NB, DH), lambda i: (0, i, 0)),
            pl.BlockSpec((NB, DH), lambda i: (i, 0)),
            full((DH, DH)), full((1, DH)), full((DH, DF)), full((1, DF)),
        ],
        out_specs=pl.BlockSpec((NB, DF), lambda i: (i, 0)),
        out_shape=jax.ShapeDtypeStruct((N, DF), jnp.float32),
    )(acc, cnt, r, W_o1, b_o1, W_o2, b_o2)


def kernel(x, edge_index, edge_attr, W_in1, b_in1, ga_in1, be_in1, W_in2, b_in2,
           ga_in2, be_in2, g0, mu0, sigma0, root0, bias0, g1, mu1, sigma1, root1,
           bias1, g2, mu2, sigma2, root2, bias2, W_o1, b_o1, W_o2, b_o2):
    src = edge_index[0]
    dst = edge_index[1]

    sc = 1.0 / jnp.sqrt(jnp.float32(1.0 + 1e-5))
    W1f = W_in1 * (ga_in1 * sc)[None, :]
    b1f = (b_in1 * ga_in1 * sc + be_in1)[None, :]
    W2f = W_in2 * (ga_in2 * sc)[None, :]
    b2f = (b_in2 * ga_in2 * sc + be_in2)[None, :]

    def pack_consts(mu, sigma):
        a = -0.5 / (1e-15 + sigma * sigma)
        A = jnp.pad(a, ((0, 2), (0, 0)))
        B = jnp.pad(-2.0 * a * mu, ((0, 2), (0, 0)))
        C = jnp.pad((a * mu * mu).sum(-1), (0, 2))
        return jnp.concatenate([A, B, C[:, None], jnp.zeros((8, 7), jnp.float32)], axis=1)

    c0 = pack_consts(mu0, sigma0)
    c1 = pack_consts(mu1, sigma1)
    c2 = pack_consts(mu2, sigma2)
    eaT = edge_attr.T

    gT0, gT1, gT2 = _gauss(eaT, c0, c1, c2)
    y, r = _lin_in(x, W1f, b1f, W2f, b2f, g0, root0, bias0[None, :])
    zeros_n = jnp.zeros((N, DH), jnp.float32)

    acc, cnt = _sc_pass_fn(True)(y, src, dst, gT0, zeros_n)
    y, r = _combine(acc, cnt, r, g1, root1, bias1[None, :])
    acc, _ = _sc_pass_fn(False)(y, src, dst, gT1, zeros_n)
    y, r = _combine(acc, cnt, r, g2, root2, bias2[None, :])
    acc, _ = _sc_pass_fn(False)(y, src, dst, gT2, zeros_n)
    out = _combine_out(acc, cnt, r, W_o1, b_o1[None, :], W_o2, b_o2[None, :])
    return out


# speedup vs baseline: 3.5575x; 1.0852x over previous
import functools

import jax
import jax.numpy as jnp
from jax import lax
from jax.experimental import pallas as pl
from jax.experimental.pallas import tpu as pltpu
from jax.experimental.pallas import tpu_sc as plsc

N = 10000
E = 320000
DF = 128
DH = 16
ED = 4
K = 6
YW = K * DH

NB = 400
GRID = N // NB
EB = 2560
NWORK = 32
BE = 128
NBLK = 80
EWP = NBLK * BE
EP = NWORK * EWP
NP = 10240
STRIPE = NP // 16


def _lin_in_body(x_ref, w1_ref, b1_ref, w2_ref, b2_ref, g_ref, rt_ref, bi_ref,
                 y_ref, r_ref):
    h = jnp.dot(x_ref[...], w1_ref[...], preferred_element_type=jnp.float32)
    h = jax.nn.relu(h + b1_ref[...])
    h = jnp.dot(h, w2_ref[...], preferred_element_type=jnp.float32)
    h = jax.nn.relu(h + b2_ref[...])
    y_ref[...] = jnp.dot(h, g_ref[...], preferred_element_type=jnp.float32)
    r_ref[...] = jnp.dot(h, rt_ref[...], preferred_element_type=jnp.float32) + bi_ref[...]


def _lin_in(x, W1f, b1f, W2f, b2f, g0, root0, bias0):
    full = lambda shape: pl.BlockSpec(shape, lambda i: (0,) * len(shape))
    return pl.pallas_call(
        _lin_in_body,
        grid=(GRID,),
        in_specs=[
            pl.BlockSpec((NB, DF), lambda i: (i, 0)),
            full((DF, DF)), full((1, DF)), full((DF, DH)), full((1, DH)),
            full((DH, YW)), full((DH, DH)), full((1, DH)),
        ],
        out_specs=[
            pl.BlockSpec((NB, YW), lambda i: (i, 0)),
            pl.BlockSpec((NB, DH), lambda i: (i, 0)),
        ],
        out_shape=[
            jax.ShapeDtypeStruct((N, YW), jnp.float32),
            jax.ShapeDtypeStruct((N, DH), jnp.float32),
        ],
    )(x, W1f, b1f, W2f, b2f, g0, root0, bias0)


def _gauss_body(ea_ref, c0_ref, c1_ref, c2_ref, o0_ref, o1_ref, o2_ref):
    ea = ea_ref[...]
    ea2 = ea * ea
    for c_ref, o_ref in ((c0_ref, o0_ref), (c1_ref, o1_ref), (c2_ref, o2_ref)):
        cc = c_ref[...]
        acc = cc[:, 8:9]
        for d in range(ED):
            acc = acc + cc[:, d:d + 1] * ea2[d:d + 1, :] + cc[:, ED + d:ED + d + 1] * ea[d:d + 1, :]
        o_ref[...] = jnp.exp(acc)


def _gauss(eaT, c0, c1, c2):
    full = lambda shape: pl.BlockSpec(shape, lambda i: (0,) * len(shape))
    ospec = pl.BlockSpec((8, EB), lambda i: (0, i))
    oshape = jax.ShapeDtypeStruct((8, EP), jnp.float32)
    return pl.pallas_call(
        _gauss_body,
        grid=(EP // EB,),
        in_specs=[pl.BlockSpec((4, EB), lambda i: (0, i)),
                  full((8, 16)), full((8, 16)), full((8, 16))],
        out_specs=[ospec, ospec, ospec],
        out_shape=[oshape, oshape, oshape],
    )(eaT, c0, c1, c2)


def _sc_body(with_cnt, *refs):
    if with_cnt:
        (y_hbm, src_hbm, dst_hbm, g_hbm, acc_out, cnt_out,
         src_v, dst_v, g_bufs, y_bufs, msg_bufs, fill_v, accum, cnt_accum,
         sem0, sem1) = refs
    else:
        (y_hbm, src_hbm, dst_hbm, g_hbm, acc_out,
         src_v, dst_v, g_bufs, y_bufs, msg_bufs, fill_v, accum,
         sem0, sem1) = refs
        cnt_out = cnt_accum = None
    c = lax.axis_index("c")
    s = lax.axis_index("s")
    w = s * 2 + c

    pltpu.sync_copy(src_hbm.at[pl.ds(w * NBLK, NBLK), :], src_v)
    pltpu.sync_copy(dst_hbm.at[pl.ds(w * NBLK, NBLK), :], dst_v)

    def fill_rows(val):
        def body(j, carry):
            fill_v[j, :] = jnp.full((16,), val, jnp.float32)
            return carry
        lax.fori_loop(0, BE, body, 0)

    fill_rows(0.0)
    for si in range(STRIPE // BE):
        pltpu.sync_copy(fill_v, accum.at[pl.ds(s * STRIPE + si * BE, BE)])
        if with_cnt:
            pltpu.sync_copy(fill_v, cnt_accum.at[pl.ds(s * STRIPE + si * BE, BE)])
    if with_cnt:
        fill_rows(1.0)
    ones_v = fill_v
    plsc.subcore_barrier()

    sems = (sem0, sem1)

    def g_slice(j):
        return g_hbm.at[pl.ds(0, K), pl.ds(w * EWP + j * BE, BE)]

    def start_fetch(j, buf_i):
        pltpu.async_copy(y_hbm.at[src_v.at[j]], y_bufs[buf_i], sems[buf_i])
        pltpu.async_copy(g_slice(j), g_bufs[buf_i], sems[buf_i])

    def wait_fetch(j, buf_i):
        pltpu.make_async_copy(y_hbm.at[src_v.at[j]], y_bufs[buf_i], sems[buf_i]).wait()
        pltpu.make_async_copy(g_slice(j), g_bufs[buf_i], sems[buf_i]).wait()

    def compute_and_scatter(j, buf_i):
        y_blk = y_bufs[buf_i]
        g_blk = g_bufs[buf_i]
        msg_v = msg_bufs[buf_i]

        def group_body(gi, carry2):
            e0 = gi * 16
            gv = [g_blk[k, pl.ds(e0, 16)] for k in range(K)]
            for jj in range(16):
                e = gi * 16 + jj
                acc = gv[0][jj] * y_blk[e, pl.ds(0, 16)]
                for k in range(1, K):
                    acc = acc + gv[k][jj] * y_blk[e, pl.ds(k * 16, 16)]
                msg_v[e, :] = acc
            return carry2
        lax.fori_loop(0, BE // 16, group_body, 0)

        pltpu.sync_copy(msg_v, accum.at[dst_v.at[j]], add=True)
        if with_cnt:
            pltpu.sync_copy(ones_v, cnt_accum.at[dst_v.at[j]], add=True)

    start_fetch(0, 0)

    def pipe_body(j2, carry):
        j = j2 * 2
        start_fetch(j + 1, 1)
        wait_fetch(j, 0)
        compute_and_scatter(j, 0)

        @pl.when(j2 < NBLK // 2 - 1)
        def _():
            start_fetch(j + 2, 0)
        wait_fetch(j + 1, 1)
        compute_and_scatter(j + 1, 1)
        return carry
    lax.fori_loop(0, NBLK // 2, pipe_body, 0)

    plsc.subcore_barrier()
    pltpu.sync_copy(accum.at[pl.ds(s * STRIPE, STRIPE)],
                    acc_out.at[c, pl.ds(s * STRIPE, STRIPE)])
    if with_cnt:
        pltpu.sync_copy(cnt_accum.at[pl.ds(s * STRIPE, STRIPE)],
                        cnt_out.at[c, pl.ds(s * STRIPE, STRIPE)])


def _make_sc(with_cnt):
    mesh = plsc.VectorSubcoreMesh(core_axis_name="c", subcore_axis_name="s",
                                  num_cores=2, num_subcores=16)
    out_type = [jax.ShapeDtypeStruct((2, NP, DH), jnp.float32)]
    if with_cnt:
        out_type.append(jax.ShapeDtypeStruct((2, NP, DH), jnp.float32))
    scratch = [
        pltpu.VMEM((NBLK, BE), jnp.int32),
        pltpu.VMEM((NBLK, BE), jnp.int32),
        [pltpu.VMEM((K, BE), jnp.float32)] * 2,
        [pltpu.VMEM((BE, YW), jnp.float32)] * 2,
        [pltpu.VMEM((BE, DH), jnp.float32)] * 2,
        pltpu.VMEM((BE, DH), jnp.float32),
        pltpu.VMEM_SHARED((NP, DH), jnp.float32),
    ]
    if with_cnt:
        scratch.append(pltpu.VMEM_SHARED((NP, DH), jnp.float32))
    scratch += [pltpu.SemaphoreType.DMA, pltpu.SemaphoreType.DMA]
    return pl.kernel(
        functools.partial(_sc_body, with_cnt),
        out_type=out_type,
        mesh=mesh,
        scratch_types=scratch,
        compiler_params=pltpu.CompilerParams(use_tc_tiling_on_sc=False),
    )


@functools.lru_cache(maxsize=None)
def _sc_pass_fn(with_cnt):
    return _make_sc(with_cnt)


def _combine_body(acc_ref, cnt_ref, r_ref, g_ref, rt_ref, bi_ref, y_ref, rn_ref):
    s = acc_ref[0] + acc_ref[1]
    c = cnt_ref[0] + cnt_ref[1]
    aggr = s / jnp.maximum(c, 1.0)
    h = jax.nn.relu(aggr + r_ref[...])
    y_ref[...] = jnp.dot(h, g_ref[...], preferred_element_type=jnp.float32)
    rn_ref[...] = jnp.dot(h, rt_ref[...], preferred_element_type=jnp.float32) + bi_ref[...]


def _combine(acc, cnt, r, g_next, root_next, bias_next):
    full = lambda shape: pl.BlockSpec(shape, lambda i: (0,) * len(shape))
    return pl.pallas_call(
        _combine_body,
        grid=(GRID,),
        in_specs=[
            pl.BlockSpec((2, NB, DH), lambda i: (0, i, 0)),
            pl.BlockSpec((2, NB, DH), lambda i: (0, i, 0)),
            pl.BlockSpec((NB, DH), lambda i: (i, 0)),
            full((DH, YW)), full((DH, DH)), full((1, DH)),
        ],
        out_specs=[
            pl.BlockSpec((NB, YW), lambda i: (i, 0)),
            pl.BlockSpec((NB, DH), lambda i: (i, 0)),
        ],
        out_shape=[
            jax.ShapeDtypeStruct((N, YW), jnp.float32),
            jax.ShapeDtypeStruct((N, DH), jnp.float32),
        ],
    )(acc, cnt, r, g_next, root_next, bias_next)


def _combine_out_body(acc_ref, cnt_ref, r_ref, w1_ref, b1_ref, w2_ref, b2_ref, o_ref):
    s = acc_ref[0] + acc_ref[1]
    c = cnt_ref[0] + cnt_ref[1]
    aggr = s / jnp.maximum(c, 1.0)
    h = jax.nn.relu(aggr + r_ref[...])
    t = jax.nn.relu(jnp.dot(h, w1_ref[...], preferred_element_type=jnp.float32) + b1_ref[...])
    o_ref[...] = jnp.dot(t, w2_ref[...], preferred_element_type=jnp.float32) + b2_ref[...]


def _combine_out(acc, cnt, r, W_o1, b_o1, W_o2, b_o2):
    full = lambda shape: pl.BlockSpec(shape, lambda i: (0,) * len(shape))
    return pl.pallas_call(
        _combine_out_body,
        grid=(GRID,),
        in_specs=[
            pl.BlockSpec((2, NB, DH), lambda i: (0, i, 0)),
            pl.BlockSpec((2, NB, DH), lambda i: (0, i, 0)),
            pl.BlockSpec((NB, DH), lambda i: (i, 0)),
            full((DH, DH)), full((1, DH)), full((DH, DF)), full((1, DF)),
        ],
        out_specs=pl.BlockSpec((NB, DF), lambda i: (i, 0)),
        out_shape=jax.ShapeDtypeStruct((N, DF), jnp.float32),
    )(acc, cnt, r, W_o1, b_o1, W_o2, b_o2)


def kernel(x, edge_index, edge_attr, W_in1, b_in1, ga_in1, be_in1, W_in2, b_in2,
           ga_in2, be_in2, g0, mu0, sigma0, root0, bias0, g1, mu1, sigma1, root1,
           bias1, g2, mu2, sigma2, root2, bias2, W_o1, b_o1, W_o2, b_o2):
    src = edge_index[0]
    dst = edge_index[1]

    sc = 1.0 / jnp.sqrt(jnp.float32(1.0 + 1e-5))
    W1f = W_in1 * (ga_in1 * sc)[None, :]
    b1f = (b_in1 * ga_in1 * sc + be_in1)[None, :]
    W2f = W_in2 * (ga_in2 * sc)[None, :]
    b2f = (b_in2 * ga_in2 * sc + be_in2)[None, :]

    def pack_consts(mu, sigma):
        a = -0.5 / (1e-15 + sigma * sigma)
        A = jnp.pad(a, ((0, 2), (0, 0)))
        B = jnp.pad(-2.0 * a * mu, ((0, 2), (0, 0)))
        C = jnp.pad((a * mu * mu).sum(-1), (0, 2))
        return jnp.concatenate([A, B, C[:, None], jnp.zeros((8, 7), jnp.float32)], axis=1)

    c0 = pack_consts(mu0, sigma0)
    c1 = pack_consts(mu1, sigma1)
    c2 = pack_consts(mu2, sigma2)
    eaT = jnp.pad(edge_attr.T, ((0, 0), (0, EP - E)))
    src_p = jnp.concatenate([src, jnp.zeros((EP - E,), jnp.int32)]).reshape(NWORK * NBLK, BE)
    dst_p = jnp.concatenate([dst, jnp.full((EP - E,), N, jnp.int32)]).reshape(NWORK * NBLK, BE)

    gT0, gT1, gT2 = _gauss(eaT, c0, c1, c2)
    y, r = _lin_in(x, W1f, b1f, W2f, b2f, g0, root0, bias0[None, :])

    acc, cnt = _sc_pass_fn(True)(y, src_p, dst_p, gT0)
    y, r = _combine(acc, cnt, r, g1, root1, bias1[None, :])
    (acc,) = _sc_pass_fn(False)(y, src_p, dst_p, gT1)
    y, r = _combine(acc, cnt, r, g2, root2, bias2[None, :])
    (acc,) = _sc_pass_fn(False)(y, src_p, dst_p, gT2)
    out = _combine_out(acc, cnt, r, W_o1, b_o1[None, :], W_o2, b_o2[None, :])
    return out


# speedup vs baseline: 5.4817x; 1.5409x over previous
import functools

import jax
import jax.numpy as jnp
import numpy as np
from jax import lax
from jax.experimental import pallas as pl
from jax.experimental.pallas import tpu as pltpu
from jax.experimental.pallas import tpu_sc as plsc

N = 10000
E = 320000
DF = 128
DH = 16
ED = 4
K = 6
YW = K * DH

NB = 400
GRID = N // NB
EB = 2560
NWORK = 32
BE = 128
NBLK = 80
EWP = NBLK * BE
EP = NWORK * EWP
NP = 10240
STRIPE = NP // 16

_PERM = np.empty((YW,), np.int32)
for _t in range(K // 2):
    _PERM[32 * _t:32 * _t + 32:2] = np.arange(16) + (2 * _t) * 16
    _PERM[32 * _t + 1:32 * _t + 32:2] = np.arange(16) + (2 * _t + 1) * 16


def _lin_in_body(x_ref, w1_ref, b1_ref, w2_ref, b2_ref, g_ref, rt_ref, bi_ref,
                 y_ref, r_ref):
    h = jnp.dot(x_ref[...], w1_ref[...], preferred_element_type=jnp.float32)
    h = jax.nn.relu(h + b1_ref[...])
    h = jnp.dot(h, w2_ref[...], preferred_element_type=jnp.float32)
    h = jax.nn.relu(h + b2_ref[...])
    y_ref[...] = jnp.dot(h, g_ref[...], preferred_element_type=jnp.float32).astype(jnp.bfloat16)
    r_ref[...] = jnp.dot(h, rt_ref[...], preferred_element_type=jnp.float32) + bi_ref[...]


def _lin_in(x, W1f, b1f, W2f, b2f, g0, root0, bias0):
    full = lambda shape: pl.BlockSpec(shape, lambda i: (0,) * len(shape))
    return pl.pallas_call(
        _lin_in_body,
        grid=(GRID,),
        in_specs=[
            pl.BlockSpec((NB, DF), lambda i: (i, 0)),
            full((DF, DF)), full((1, DF)), full((DF, DH)), full((1, DH)),
            full((DH, YW)), full((DH, DH)), full((1, DH)),
        ],
        out_specs=[
            pl.BlockSpec((NB, YW), lambda i: (i, 0)),
            pl.BlockSpec((NB, DH), lambda i: (i, 0)),
        ],
        out_shape=[
            jax.ShapeDtypeStruct((N, YW), jnp.bfloat16),
            jax.ShapeDtypeStruct((N, DH), jnp.float32),
        ],
    )(x, W1f, b1f, W2f, b2f, g0, root0, bias0)


def _gauss_body(ea_ref, c0_ref, c1_ref, c2_ref, o0_ref, o1_ref, o2_ref):
    ea = ea_ref[...]
    ea2 = ea * ea
    for c_ref, o_ref in ((c0_ref, o0_ref), (c1_ref, o1_ref), (c2_ref, o2_ref)):
        cc = c_ref[...]
        acc = cc[:, 8:9]
        for d in range(ED):
            acc = acc + cc[:, d:d + 1] * ea2[d:d + 1, :] + cc[:, ED + d:ED + d + 1] * ea[d:d + 1, :]
        o_ref[...] = jnp.exp(acc)


def _gauss(eaT, c0, c1, c2):
    full = lambda shape: pl.BlockSpec(shape, lambda i: (0,) * len(shape))
    ospec = pl.BlockSpec((8, EB), lambda i: (0, i))
    oshape = jax.ShapeDtypeStruct((8, EP), jnp.float32)
    return pl.pallas_call(
        _gauss_body,
        grid=(EP // EB,),
        in_specs=[pl.BlockSpec((4, EB), lambda i: (0, i)),
                  full((8, 16)), full((8, 16)), full((8, 16))],
        out_specs=[ospec, ospec, ospec],
        out_shape=[oshape, oshape, oshape],
    )(eaT, c0, c1, c2)


def _sc_body(with_cnt, *refs):
    if with_cnt:
        (y_hbm, src_hbm, dst_hbm, g_hbm, acc_out, cnt_out,
         src_v, dst_v, g_bufs, y_bufs, msg_bufs, fill_v, accum, cnt_accum,
         sem0, sem1) = refs
    else:
        (y_hbm, src_hbm, dst_hbm, g_hbm, acc_out,
         src_v, dst_v, g_bufs, y_bufs, msg_bufs, fill_v, accum,
         sem0, sem1) = refs
        cnt_out = cnt_accum = None
    c = lax.axis_index("c")
    s = lax.axis_index("s")
    w = s * 2 + c

    pltpu.sync_copy(src_hbm.at[pl.ds(w * NBLK, NBLK), :], src_v)
    pltpu.sync_copy(dst_hbm.at[pl.ds(w * NBLK, NBLK), :], dst_v)

    def fill_rows(val):
        def body(j, carry):
            fill_v[j, :] = jnp.full((16,), val, jnp.float32)
            return carry
        lax.fori_loop(0, BE, body, 0)

    fill_rows(0.0)
    for si in range(STRIPE // BE):
        pltpu.sync_copy(fill_v, accum.at[pl.ds(s * STRIPE + si * BE, BE)])
        if with_cnt:
            pltpu.sync_copy(fill_v, cnt_accum.at[pl.ds(s * STRIPE + si * BE, BE)])
    if with_cnt:
        fill_rows(1.0)
    ones_v = fill_v
    plsc.subcore_barrier()

    sems = (sem0, sem1)

    def g_slice(j):
        return g_hbm.at[pl.ds(0, K), pl.ds(w * EWP + j * BE, BE)]

    def start_fetch(j, buf_i):
        pltpu.async_copy(y_hbm.at[src_v.at[j]], y_bufs[buf_i], sems[buf_i])
        pltpu.async_copy(g_slice(j), g_bufs[buf_i], sems[buf_i])

    def wait_fetch(j, buf_i):
        pltpu.make_async_copy(y_hbm.at[src_v.at[j]], y_bufs[buf_i], sems[buf_i]).wait()
        pltpu.make_async_copy(g_slice(j), g_bufs[buf_i], sems[buf_i]).wait()

    def compute_and_scatter(j, buf_i):
        y_blk = y_bufs[buf_i]
        g_blk = g_bufs[buf_i]
        msg_v = msg_bufs[buf_i]

        def group_body(gi, carry2):
            e0 = gi * 16
            gv = [g_blk[k, pl.ds(e0, 16)] for k in range(K)]
            for jj in range(16):
                e = gi * 16 + jj
                acc = None
                for t in range(K // 2):
                    p = y_blk[e, pl.ds(16 * t, 16)]
                    a = lax.bitcast_convert_type(jnp.left_shift(p, 16), jnp.float32)
                    b = lax.bitcast_convert_type(jnp.bitwise_and(p, jnp.int32(-65536)), jnp.float32)
                    term = gv[2 * t][jj] * a + gv[2 * t + 1][jj] * b
                    acc = term if acc is None else acc + term
                msg_v[e, :] = acc
            return carry2
        lax.fori_loop(0, BE // 16, group_body, 0)

        pltpu.sync_copy(msg_v, accum.at[dst_v.at[j]], add=True)
        if with_cnt:
            pltpu.sync_copy(ones_v, cnt_accum.at[dst_v.at[j]], add=True)

    start_fetch(0, 0)

    def pipe_body(j2, carry):
        j = j2 * 2
        start_fetch(j + 1, 1)
        wait_fetch(j, 0)
        compute_and_scatter(j, 0)

        @pl.when(j2 < NBLK // 2 - 1)
        def _():
            start_fetch(j + 2, 0)
        wait_fetch(j + 1, 1)
        compute_and_scatter(j + 1, 1)
        return carry
    lax.fori_loop(0, NBLK // 2, pipe_body, 0)

    plsc.subcore_barrier()
    pltpu.sync_copy(accum.at[pl.ds(s * STRIPE, STRIPE)],
                    acc_out.at[c, pl.ds(s * STRIPE, STRIPE)])
    if with_cnt:
        pltpu.sync_copy(cnt_accum.at[pl.ds(s * STRIPE, STRIPE)],
                        cnt_out.at[c, pl.ds(s * STRIPE, STRIPE)])


def _make_sc(with_cnt):
    mesh = plsc.VectorSubcoreMesh(core_axis_name="c", subcore_axis_name="s",
                                  num_cores=2, num_subcores=16)
    out_type = [jax.ShapeDtypeStruct((2, NP, DH), jnp.float32)]
    if with_cnt:
        out_type.append(jax.ShapeDtypeStruct((2, NP, DH), jnp.float32))
    scratch = [
        pltpu.VMEM((NBLK, BE), jnp.int32),
        pltpu.VMEM((NBLK, BE), jnp.int32),
        [pltpu.VMEM((K, BE), jnp.float32)] * 2,
        [pltpu.VMEM((BE, YW // 2), jnp.int32)] * 2,
        [pltpu.VMEM((BE, DH), jnp.float32)] * 2,
        pltpu.VMEM((BE, DH), jnp.float32),
        pltpu.VMEM_SHARED((NP, DH), jnp.float32),
    ]
    if with_cnt:
        scratch.append(pltpu.VMEM_SHARED((NP, DH), jnp.float32))
    scratch += [pltpu.SemaphoreType.DMA, pltpu.SemaphoreType.DMA]
    return pl.kernel(
        functools.partial(_sc_body, with_cnt),
        out_type=out_type,
        mesh=mesh,
        scratch_types=scratch,
        compiler_params=pltpu.CompilerParams(use_tc_tiling_on_sc=False),
    )


@functools.lru_cache(maxsize=None)
def _sc_pass_fn(with_cnt):
    return _make_sc(with_cnt)


def _combine_body(acc_ref, cnt_ref, r_ref, g_ref, rt_ref, bi_ref, y_ref, rn_ref):
    s = acc_ref[0] + acc_ref[1]
    c = cnt_ref[0] + cnt_ref[1]
    aggr = s / jnp.maximum(c, 1.0)
    h = jax.nn.relu(aggr + r_ref[...])
    y_ref[...] = jnp.dot(h, g_ref[...], preferred_element_type=jnp.float32).astype(jnp.bfloat16)
    rn_ref[...] = jnp.dot(h, rt_ref[...], preferred_element_type=jnp.float32) + bi_ref[...]


def _combine(acc, cnt, r, g_next, root_next, bias_next):
    full = lambda shape: pl.BlockSpec(shape, lambda i: (0,) * len(shape))
    return pl.pallas_call(
        _combine_body,
        grid=(GRID,),
        in_specs=[
            pl.BlockSpec((2, NB, DH), lambda i: (0, i, 0)),
            pl.BlockSpec((2, NB, DH), lambda i: (0, i, 0)),
            pl.BlockSpec((NB, DH), lambda i: (i, 0)),
            full((DH, YW)), full((DH, DH)), full((1, DH)),
        ],
        out_specs=[
            pl.BlockSpec((NB, YW), lambda i: (i, 0)),
            pl.BlockSpec((NB, DH), lambda i: (i, 0)),
        ],
        out_shape=[
            jax.ShapeDtypeStruct((N, YW), jnp.bfloat16),
            jax.ShapeDtypeStruct((N, DH), jnp.float32),
        ],
    )(acc, cnt, r, g_next, root_next, bias_next)


def _combine_out_body(acc_ref, cnt_ref, r_ref, w1_ref, b1_ref, w2_ref, b2_ref, o_ref):
    s = acc_ref[0] + acc_ref[1]
    c = cnt_ref[0] + cnt_ref[1]
    aggr = s / jnp.maximum(c, 1.0)
    h = jax.nn.relu(aggr + r_ref[...])
    t = jax.nn.relu(jnp.dot(h, w1_ref[...], preferred_element_type=jnp.float32) + b1_ref[...])
    o_ref[...] = jnp.dot(t, w2_ref[...], preferred_element_type=jnp.float32) + b2_ref[...]


def _combine_out(acc, cnt, r, W_o1, b_o1, W_o2, b_o2):
    full = lambda shape: pl.BlockSpec(shape, lambda i: (0,) * len(shape))
    return pl.pallas_call(
        _combine_out_body,
        grid=(GRID,),
        in_specs=[
            pl.BlockSpec((2, NB, DH), lambda i: (0, i, 0)),
            pl.BlockSpec((2, NB, DH), lambda i: (0, i, 0)),
            pl.BlockSpec((NB, DH), lambda i: (i, 0)),
            full((DH, DH)), full((1, DH)), full((DH, DF)), full((1, DF)),
        ],
        out_specs=pl.BlockSpec((NB, DF), lambda i: (i, 0)),
        out_shape=jax.ShapeDtypeStruct((N, DF), jnp.float32),
    )(acc, cnt, r, W_o1, b_o1, W_o2, b_o2)


def kernel(x, edge_index, edge_attr, W_in1, b_in1, ga_in1, be_in1, W_in2, b_in2,
           ga_in2, be_in2, g0, mu0, sigma0, root0, bias0, g1, mu1, sigma1, root1,
           bias1, g2, mu2, sigma2, root2, bias2, W_o1, b_o1, W_o2, b_o2):
    src = edge_index[0]
    dst = edge_index[1]

    sc = 1.0 / jnp.sqrt(jnp.float32(1.0 + 1e-5))
    W1f = W_in1 * (ga_in1 * sc)[None, :]
    b1f = (b_in1 * ga_in1 * sc + be_in1)[None, :]
    W2f = W_in2 * (ga_in2 * sc)[None, :]
    b2f = (b_in2 * ga_in2 * sc + be_in2)[None, :]

    def pack_consts(mu, sigma):
        a = -0.5 / (1e-15 + sigma * sigma)
        A = jnp.pad(a, ((0, 2), (0, 0)))
        B = jnp.pad(-2.0 * a * mu, ((0, 2), (0, 0)))
        C = jnp.pad((a * mu * mu).sum(-1), (0, 2))
        return jnp.concatenate([A, B, C[:, None], jnp.zeros((8, 7), jnp.float32)], axis=1)

    c0 = pack_consts(mu0, sigma0)
    c1 = pack_consts(mu1, sigma1)
    c2 = pack_consts(mu2, sigma2)
    eaT = jnp.pad(edge_attr.T, ((0, 0), (0, EP - E)))
    src_p = jnp.concatenate([src, jnp.zeros((EP - E,), jnp.int32)]).reshape(NWORK * NBLK, BE)
    dst_p = jnp.concatenate([dst, jnp.full((EP - E,), N, jnp.int32)]).reshape(NWORK * NBLK, BE)

    gT0, gT1, gT2 = _gauss(eaT, c0, c1, c2)
    perm = jnp.asarray(_PERM)
    g0p, g1p, g2p = g0[:, perm], g1[:, perm], g2[:, perm]
    y, r = _lin_in(x, W1f, b1f, W2f, b2f, g0p, root0, bias0[None, :])

    def pack_y(yb):
        return lax.bitcast_convert_type(yb.reshape(N, YW // 2, 2), jnp.int32)

    acc, cnt = _sc_pass_fn(True)(pack_y(y), src_p, dst_p, gT0)
    y, r = _combine(acc, cnt, r, g1p, root1, bias1[None, :])
    (acc,) = _sc_pass_fn(False)(pack_y(y), src_p, dst_p, gT1)
    y, r = _combine(acc, cnt, r, g2p, root2, bias2[None, :])
    (acc,) = _sc_pass_fn(False)(pack_y(y), src_p, dst_p, gT2)
    out = _combine_out(acc, cnt, r, W_o1, b_o1[None, :], W_o2, b_o2[None, :])
    return out


# speedup vs baseline: 6.0491x; 1.1035x over previous
import functools

import jax
import jax.numpy as jnp
import numpy as np
from jax import lax
from jax.experimental import pallas as pl
from jax.experimental.pallas import tpu as pltpu
from jax.experimental.pallas import tpu_sc as plsc

N = 10000
E = 320000
DF = 128
DH = 16
ED = 4
K = 6
YW = K * DH

NB = 400
GRID = N // NB
EB = 2560
NWORK = 32
BE = 128
NBLK0 = 96
NBLK1 = 64
NBLKT = 16 * NBLK0 + 16 * NBLK1
EP = NBLKT * BE
NP = 10240
STRIPE = NP // 16
YP = YW // 2

_COLS_A = np.concatenate([np.arange(16) + (2 * t) * 16 for t in range(K // 2)])
_COLS_B = np.concatenate([np.arange(16) + (2 * t + 1) * 16 for t in range(K // 2)])


def _pack_pair(ya, yb):
    ua = lax.bitcast_convert_type(ya.astype(jnp.bfloat16), jnp.uint16).astype(jnp.int32)
    ub = lax.bitcast_convert_type(yb.astype(jnp.bfloat16), jnp.uint16).astype(jnp.int32)
    return ua | (ub << 16)


def _lin_in_body(x_ref, w1_ref, b1_ref, w2_ref, b2_ref, ga_ref, gb_ref, rt_ref,
                 bi_ref, y_ref, r_ref):
    h = jnp.dot(x_ref[...], w1_ref[...], preferred_element_type=jnp.float32)
    h = jax.nn.relu(h + b1_ref[...])
    h = jnp.dot(h, w2_ref[...], preferred_element_type=jnp.float32)
    h = jax.nn.relu(h + b2_ref[...])
    ya = jnp.dot(h, ga_ref[...], preferred_element_type=jnp.float32)
    yb = jnp.dot(h, gb_ref[...], preferred_element_type=jnp.float32)
    y_ref[...] = _pack_pair(ya, yb)
    r_ref[...] = jnp.dot(h, rt_ref[...], preferred_element_type=jnp.float32) + bi_ref[...]


def _lin_in(x, W1f, b1f, W2f, b2f, ga, gb, root0, bias0):
    full = lambda shape: pl.BlockSpec(shape, lambda i: (0,) * len(shape))
    return pl.pallas_call(
        _lin_in_body,
        grid=(GRID,),
        in_specs=[
            pl.BlockSpec((NB, DF), lambda i: (i, 0)),
            full((DF, DF)), full((1, DF)), full((DF, DH)), full((1, DH)),
            full((DH, YP)), full((DH, YP)), full((DH, DH)), full((1, DH)),
        ],
        out_specs=[
            pl.BlockSpec((NB, YP), lambda i: (i, 0)),
            pl.BlockSpec((NB, DH), lambda i: (i, 0)),
        ],
        out_shape=[
            jax.ShapeDtypeStruct((N, YP), jnp.int32),
            jax.ShapeDtypeStruct((N, DH), jnp.float32),
        ],
    )(x, W1f, b1f, W2f, b2f, ga, gb, root0, bias0)


def _gauss_body(ea_ref, c0_ref, c1_ref, c2_ref, o0_ref, o1_ref, o2_ref):
    ea = ea_ref[...]
    ea2 = ea * ea
    for c_ref, o_ref in ((c0_ref, o0_ref), (c1_ref, o1_ref), (c2_ref, o2_ref)):
        cc = c_ref[...]
        acc = cc[:, 8:9]
        for d in range(ED):
            acc = acc + cc[:, d:d + 1] * ea2[d:d + 1, :] + cc[:, ED + d:ED + d + 1] * ea[d:d + 1, :]
        o_ref[...] = jnp.exp(acc)


def _gauss(eaT, c0, c1, c2):
    full = lambda shape: pl.BlockSpec(shape, lambda i: (0,) * len(shape))
    ospec = pl.BlockSpec((8, EB), lambda i: (0, i))
    oshape = jax.ShapeDtypeStruct((8, EP), jnp.float32)
    return pl.pallas_call(
        _gauss_body,
        grid=(EP // EB,),
        in_specs=[pl.BlockSpec((4, EB), lambda i: (0, i)),
                  full((8, 16)), full((8, 16)), full((8, 16))],
        out_specs=[ospec, ospec, ospec],
        out_shape=[oshape, oshape, oshape],
    )(eaT, c0, c1, c2)


def _sc_body(with_cnt, *refs):
    if with_cnt:
        (y_hbm, src_hbm, dst_hbm, g_hbm, acc_out, cnt_out,
         src_v, dst_v, g_bufs, y_bufs, msg_bufs, fill_v, accum, cnt_accum,
         sem0, sem1) = refs
    else:
        (y_hbm, src_hbm, dst_hbm, g_hbm, acc_out,
         src_v, dst_v, g_bufs, y_bufs, msg_bufs, fill_v, accum,
         sem0, sem1) = refs
        cnt_out = cnt_accum = None
    c = lax.axis_index("c")
    s = lax.axis_index("s")
    w = c * 16 + s
    nb2 = jnp.where(c == 0, NBLK0 // 2, NBLK1 // 2)
    gb0 = jnp.where(c == 0, s * NBLK0, 16 * NBLK0 + s * NBLK1)

    pltpu.sync_copy(src_hbm.at[pl.ds(w * NBLK0, NBLK0), :], src_v)
    pltpu.sync_copy(dst_hbm.at[pl.ds(w * NBLK0, NBLK0), :], dst_v)

    def fill_rows(val):
        def body(j, carry):
            fill_v[j, :] = jnp.full((16,), val, jnp.float32)
            return carry
        lax.fori_loop(0, BE, body, 0)

    fill_rows(0.0)
    for si in range(STRIPE // BE):
        pltpu.sync_copy(fill_v, accum.at[pl.ds(s * STRIPE + si * BE, BE)])
        if with_cnt:
            pltpu.sync_copy(fill_v, cnt_accum.at[pl.ds(s * STRIPE + si * BE, BE)])
    if with_cnt:
        fill_rows(1.0)
    ones_v = fill_v
    plsc.subcore_barrier()

    sems = (sem0, sem1)

    def g_slice(j):
        return g_hbm.at[pl.ds(0, K), pl.ds((gb0 + j) * BE, BE)]

    def start_fetch(j, buf_i):
        pltpu.async_copy(y_hbm.at[src_v.at[j]], y_bufs[buf_i], sems[buf_i])
        pltpu.async_copy(g_slice(j), g_bufs[buf_i], sems[buf_i])

    def wait_fetch(j, buf_i):
        pltpu.make_async_copy(y_hbm.at[src_v.at[j]], y_bufs[buf_i], sems[buf_i]).wait()
        pltpu.make_async_copy(g_slice(j), g_bufs[buf_i], sems[buf_i]).wait()

    def compute_and_scatter(j, buf_i):
        y_blk = y_bufs[buf_i]
        g_blk = g_bufs[buf_i]
        msg_v = msg_bufs[buf_i]

        def group_body(gi, carry2):
            e0 = gi * 16
            gv = [g_blk[k, pl.ds(e0, 16)] for k in range(K)]
            for jj in range(16):
                e = gi * 16 + jj
                acc = None
                for t in range(K // 2):
                    p = y_blk[e, pl.ds(16 * t, 16)]
                    a = lax.bitcast_convert_type(jnp.left_shift(p, 16), jnp.float32)
                    b = lax.bitcast_convert_type(jnp.bitwise_and(p, jnp.int32(-65536)), jnp.float32)
                    term = gv[2 * t][jj] * a + gv[2 * t + 1][jj] * b
                    acc = term if acc is None else acc + term
                msg_v[e, :] = acc
            return carry2
        lax.fori_loop(0, BE // 16, group_body, 0)

        pltpu.sync_copy(msg_v, accum.at[dst_v.at[j]], add=True)
        if with_cnt:
            pltpu.sync_copy(ones_v, cnt_accum.at[dst_v.at[j]], add=True)

    start_fetch(0, 0)

    def pipe_body(j2, carry):
        j = j2 * 2
        start_fetch(j + 1, 1)
        wait_fetch(j, 0)
        compute_and_scatter(j, 0)

        @pl.when(j2 < nb2 - 1)
        def _():
            start_fetch(j + 2, 0)
        wait_fetch(j + 1, 1)
        compute_and_scatter(j + 1, 1)
        return carry
    lax.fori_loop(0, nb2, pipe_body, 0)

    plsc.subcore_barrier()
    pltpu.sync_copy(accum.at[pl.ds(s * STRIPE, STRIPE)],
                    acc_out.at[c, pl.ds(s * STRIPE, STRIPE)])
    if with_cnt:
        pltpu.sync_copy(cnt_accum.at[pl.ds(s * STRIPE, STRIPE)],
                        cnt_out.at[c, pl.ds(s * STRIPE, STRIPE)])


def _make_sc(with_cnt):
    mesh = plsc.VectorSubcoreMesh(core_axis_name="c", subcore_axis_name="s",
                                  num_cores=2, num_subcores=16)
    out_type = [jax.ShapeDtypeStruct((2, NP, DH), jnp.float32)]
    if with_cnt:
        out_type.append(jax.ShapeDtypeStruct((2, NP, DH), jnp.float32))
    scratch = [
        pltpu.VMEM((NBLK0, BE), jnp.int32),
        pltpu.VMEM((NBLK0, BE), jnp.int32),
        [pltpu.VMEM((K, BE), jnp.float32)] * 2,
        [pltpu.VMEM((BE, YP), jnp.int32)] * 2,
        [pltpu.VMEM((BE, DH), jnp.float32)] * 2,
        pltpu.VMEM((BE, DH), jnp.float32),
        pltpu.VMEM_SHARED((NP, DH), jnp.float32),
    ]
    if with_cnt:
        scratch.append(pltpu.VMEM_SHARED((NP, DH), jnp.float32))
    scratch += [pltpu.SemaphoreType.DMA, pltpu.SemaphoreType.DMA]
    return pl.kernel(
        functools.partial(_sc_body, with_cnt),
        out_type=out_type,
        mesh=mesh,
        scratch_types=scratch,
        compiler_params=pltpu.CompilerParams(use_tc_tiling_on_sc=False),
    )


@functools.lru_cache(maxsize=None)
def _sc_pass_fn(with_cnt):
    return _make_sc(with_cnt)


def _combine_body(acc_ref, cnt_ref, r_ref, ga_ref, gb_ref, rt_ref, bi_ref,
                  y_ref, rn_ref):
    s = acc_ref[0] + acc_ref[1]
    c = cnt_ref[0] + cnt_ref[1]
    aggr = s / jnp.maximum(c, 1.0)
    h = jax.nn.relu(aggr + r_ref[...])
    ya = jnp.dot(h, ga_ref[...], preferred_element_type=jnp.float32)
    yb = jnp.dot(h, gb_ref[...], preferred_element_type=jnp.float32)
    y_ref[...] = _pack_pair(ya, yb)
    rn_ref[...] = jnp.dot(h, rt_ref[...], preferred_element_type=jnp.float32) + bi_ref[...]


def _combine(acc, cnt, r, ga, gb, root_next, bias_next):
    full = lambda shape: pl.BlockSpec(shape, lambda i: (0,) * len(shape))
    return pl.pallas_call(
        _combine_body,
        grid=(GRID,),
        in_specs=[
            pl.BlockSpec((2, NB, DH), lambda i: (0, i, 0)),
            pl.BlockSpec((2, NB, DH), lambda i: (0, i, 0)),
            pl.BlockSpec((NB, DH), lambda i: (i, 0)),
            full((DH, YP)), full((DH, YP)), full((DH, DH)), full((1, DH)),
        ],
        out_specs=[
            pl.BlockSpec((NB, YP), lambda i: (i, 0)),
            pl.BlockSpec((NB, DH), lambda i: (i, 0)),
        ],
        out_shape=[
            jax.ShapeDtypeStruct((N, YP), jnp.int32),
            jax.ShapeDtypeStruct((N, DH), jnp.float32),
        ],
    )(acc, cnt, r, ga, gb, root_next, bias_next)


def _combine_out_body(acc_ref, cnt_ref, r_ref, w1_ref, b1_ref, w2_ref, b2_ref, o_ref):
    s = acc_ref[0] + acc_ref[1]
    c = cnt_ref[0] + cnt_ref[1]
    aggr = s / jnp.maximum(c, 1.0)
    h = jax.nn.relu(aggr + r_ref[...])
    t = jax.nn.relu(jnp.dot(h, w1_ref[...], preferred_element_type=jnp.float32) + b1_ref[...])
    o_ref[...] = jnp.dot(t, w2_ref[...], preferred_element_type=jnp.float32) + b2_ref[...]


def _combine_out(acc, cnt, r, W_o1, b_o1, W_o2, b_o2):
    full = lambda shape: pl.BlockSpec(shape, lambda i: (0,) * len(shape))
    return pl.pallas_call(
        _combine_out_body,
        grid=(GRID,),
        in_specs=[
            pl.BlockSpec((2, NB, DH), lambda i: (0, i, 0)),
            pl.BlockSpec((2, NB, DH), lambda i: (0, i, 0)),
            pl.BlockSpec((NB, DH), lambda i: (i, 0)),
            full((DH, DH)), full((1, DH)), full((DH, DF)), full((1, DF)),
        ],
        out_specs=pl.BlockSpec((NB, DF), lambda i: (i, 0)),
        out_shape=jax.ShapeDtypeStruct((N, DF), jnp.float32),
    )(acc, cnt, r, W_o1, b_o1, W_o2, b_o2)


def kernel(x, edge_index, edge_attr, W_in1, b_in1, ga_in1, be_in1, W_in2, b_in2,
           ga_in2, be_in2, g0, mu0, sigma0, root0, bias0, g1, mu1, sigma1, root1,
           bias1, g2, mu2, sigma2, root2, bias2, W_o1, b_o1, W_o2, b_o2):
    src = edge_index[0]
    dst = edge_index[1]

    sc = 1.0 / jnp.sqrt(jnp.float32(1.0 + 1e-5))
    W1f = W_in1 * (ga_in1 * sc)[None, :]
    b1f = (b_in1 * ga_in1 * sc + be_in1)[None, :]
    W2f = W_in2 * (ga_in2 * sc)[None, :]
    b2f = (b_in2 * ga_in2 * sc + be_in2)[None, :]

    def pack_consts(mu, sigma):
        a = -0.5 / (1e-15 + sigma * sigma)
        A = jnp.pad(a, ((0, 2), (0, 0)))
        B = jnp.pad(-2.0 * a * mu, ((0, 2), (0, 0)))
        C = jnp.pad((a * mu * mu).sum(-1), (0, 2))
        return jnp.concatenate([A, B, C[:, None], jnp.zeros((8, 7), jnp.float32)], axis=1)

    c0 = pack_consts(mu0, sigma0)
    c1 = pack_consts(mu1, sigma1)
    c2 = pack_consts(mu2, sigma2)
    eaT = jnp.pad(edge_attr.T, ((0, 0), (0, EP - E)))

    def make_table(v, padval):
        blk = jnp.concatenate(
            [v, jnp.full((EP - E,), padval, jnp.int32)]).reshape(NBLKT, BE)
        sc0 = blk[:16 * NBLK0].reshape(16, NBLK0, BE)
        sc1 = blk[16 * NBLK0:].reshape(16, NBLK1, BE)
        sc1 = jnp.pad(sc1, ((0, 0), (0, NBLK0 - NBLK1), (0, 0)),
                      constant_values=padval)
        return jnp.concatenate([sc0, sc1], 0).reshape(32 * NBLK0, BE)

    src_p = make_table(src, 0)
    dst_p = make_table(dst, N)

    gT0, gT1, gT2 = _gauss(eaT, c0, c1, c2)
    ca, cb = jnp.asarray(_COLS_A), jnp.asarray(_COLS_B)
    y, r = _lin_in(x, W1f, b1f, W2f, b2f, g0[:, ca], g0[:, cb], root0, bias0[None, :])

    acc, cnt = _sc_pass_fn(True)(y, src_p, dst_p, gT0)
    y, r = _combine(acc, cnt, r, g1[:, ca], g1[:, cb], root1, bias1[None, :])
    (acc,) = _sc_pass_fn(False)(y, src_p, dst_p, gT1)
    y, r = _combine(acc, cnt, r, g2[:, ca], g2[:, cb], root2, bias2[None, :])
    (acc,) = _sc_pass_fn(False)(y, src_p, dst_p, gT2)
    out = _combine_out(acc, cnt, r, W_o1, b_o1[None, :], W_o2, b_o2[None, :])
    return out


# speedup vs baseline: 6.0772x; 1.0046x over previous
import functools

import jax
import jax.numpy as jnp
import numpy as np
from jax import lax
from jax.experimental import pallas as pl
from jax.experimental.pallas import tpu as pltpu
from jax.experimental.pallas import tpu_sc as plsc

N = 10000
E = 320000
DF = 128
DH = 16
ED = 4
K = 6
YW = K * DH

NB = 400
GRID = N // NB
EB = 2560
NWORK = 32
BE = 128
NBLK0 = 96
NBLK1 = 64
NBLKT = 16 * NBLK0 + 16 * NBLK1
EP = NBLKT * BE
NP = 10240
STRIPE = NP // 16
YP = YW // 2

_COLS_A = np.concatenate([np.arange(16) + (2 * t) * 16 for t in range(K // 2)])
_COLS_B = np.concatenate([np.arange(16) + (2 * t + 1) * 16 for t in range(K // 2)])


def _pack_pair(ya, yb):
    ua = lax.bitcast_convert_type(ya.astype(jnp.bfloat16), jnp.uint16).astype(jnp.int32)
    ub = lax.bitcast_convert_type(yb.astype(jnp.bfloat16), jnp.uint16).astype(jnp.int32)
    return ua | (ub << 16)


def _lin_in_body(x_ref, w1_ref, b1_ref, w2_ref, b2_ref, ga_ref, gb_ref, rt_ref,
                 bi_ref, y_ref, r_ref):
    h = jnp.dot(x_ref[...], w1_ref[...], preferred_element_type=jnp.float32)
    h = jax.nn.relu(h + b1_ref[...])
    h = jnp.dot(h, w2_ref[...], preferred_element_type=jnp.float32)
    h = jax.nn.relu(h + b2_ref[...])
    ya = jnp.dot(h, ga_ref[...], preferred_element_type=jnp.float32)
    yb = jnp.dot(h, gb_ref[...], preferred_element_type=jnp.float32)
    y_ref[...] = _pack_pair(ya, yb)
    r_ref[...] = jnp.dot(h, rt_ref[...], preferred_element_type=jnp.float32) + bi_ref[...]


def _lin_in(x, W1f, b1f, W2f, b2f, ga, gb, root0, bias0):
    full = lambda shape: pl.BlockSpec(shape, lambda i: (0,) * len(shape))
    return pl.pallas_call(
        _lin_in_body,
        grid=(GRID,),
        in_specs=[
            pl.BlockSpec((NB, DF), lambda i: (i, 0)),
            full((DF, DF)), full((1, DF)), full((DF, DH)), full((1, DH)),
            full((DH, YP)), full((DH, YP)), full((DH, DH)), full((1, DH)),
        ],
        out_specs=[
            pl.BlockSpec((NB, YP), lambda i: (i, 0)),
            pl.BlockSpec((NB, DH), lambda i: (i, 0)),
        ],
        out_shape=[
            jax.ShapeDtypeStruct((N, YP), jnp.int32),
            jax.ShapeDtypeStruct((N, DH), jnp.float32),
        ],
    )(x, W1f, b1f, W2f, b2f, ga, gb, root0, bias0)


def _gauss_body(ea_ref, c_ref, o_ref):
    ea = ea_ref[...]
    ea2 = ea * ea
    cc = c_ref[...]
    acc = cc[:, 8:9]
    for d in range(ED):
        acc = acc + cc[:, d:d + 1] * ea2[d:d + 1, :] + cc[:, ED + d:ED + d + 1] * ea[d:d + 1, :]
    o_ref[...] = jnp.exp(acc)


def _gauss(eaT, c):
    full = lambda shape: pl.BlockSpec(shape, lambda i: (0,) * len(shape))
    return pl.pallas_call(
        _gauss_body,
        grid=(EP // EB,),
        in_specs=[pl.BlockSpec((4, EB), lambda i: (0, i)), full((8, 16))],
        out_specs=pl.BlockSpec((8, EB), lambda i: (0, i)),
        out_shape=jax.ShapeDtypeStruct((8, EP), jnp.float32),
    )(eaT, c)


def _sc_body(with_cnt, *refs):
    if with_cnt:
        (y_hbm, src_hbm, dst_hbm, g_hbm, acc_out, cnt_out,
         src_v, dst_v, g_bufs, y_bufs, msg_bufs, fill_v, accum, cnt_accum,
         sem0, sem1, ssem0, ssem1) = refs
    else:
        (y_hbm, src_hbm, dst_hbm, g_hbm, acc_out,
         src_v, dst_v, g_bufs, y_bufs, msg_bufs, fill_v, accum,
         sem0, sem1, ssem0, ssem1) = refs
        cnt_out = cnt_accum = None
    c = lax.axis_index("c")
    s = lax.axis_index("s")
    w = c * 16 + s
    nb2 = jnp.where(c == 0, NBLK0 // 2, NBLK1 // 2)
    gb0 = jnp.where(c == 0, s * NBLK0, 16 * NBLK0 + s * NBLK1)

    pltpu.sync_copy(src_hbm.at[pl.ds(w * NBLK0, NBLK0), :], src_v)
    pltpu.sync_copy(dst_hbm.at[pl.ds(w * NBLK0, NBLK0), :], dst_v)

    def fill_rows(val):
        def body(j, carry):
            fill_v[j, :] = jnp.full((16,), val, jnp.float32)
            return carry
        lax.fori_loop(0, BE, body, 0)

    fill_rows(0.0)
    for si in range(STRIPE // BE):
        pltpu.sync_copy(fill_v, accum.at[pl.ds(s * STRIPE + si * BE, BE)])
        if with_cnt:
            pltpu.sync_copy(fill_v, cnt_accum.at[pl.ds(s * STRIPE + si * BE, BE)])
    if with_cnt:
        fill_rows(1.0)
    ones_v = fill_v
    plsc.subcore_barrier()

    sems = (sem0, sem1)
    ssems = (ssem0, ssem1)

    def g_slice(j):
        return g_hbm.at[pl.ds(0, K), pl.ds((gb0 + j) * BE, BE)]

    def start_fetch(j, buf_i):
        pltpu.async_copy(y_hbm.at[src_v.at[j]], y_bufs[buf_i], sems[buf_i])
        pltpu.async_copy(g_slice(j), g_bufs[buf_i], sems[buf_i])

    def wait_fetch(j, buf_i):
        pltpu.make_async_copy(y_hbm.at[src_v.at[j]], y_bufs[buf_i], sems[buf_i]).wait()
        pltpu.make_async_copy(g_slice(j), g_bufs[buf_i], sems[buf_i]).wait()

    def compute_block(j, buf_i):
        y_blk = y_bufs[buf_i]
        g_blk = g_bufs[buf_i]
        msg_v = msg_bufs[buf_i]

        def group_body(gi, carry2):
            e0 = gi * 16
            gv = [g_blk[k, pl.ds(e0, 16)] for k in range(K)]
            for jj in range(16):
                e = gi * 16 + jj
                acc = None
                for t in range(K // 2):
                    p = y_blk[e, pl.ds(16 * t, 16)]
                    a = lax.bitcast_convert_type(jnp.left_shift(p, 16), jnp.float32)
                    b = lax.bitcast_convert_type(jnp.bitwise_and(p, jnp.int32(-65536)), jnp.float32)
                    term = gv[2 * t][jj] * a + gv[2 * t + 1][jj] * b
                    acc = term if acc is None else acc + term
                msg_v[e, :] = acc
            return carry2
        lax.fori_loop(0, BE // 16, group_body, 0)

    def start_scatter(j, buf_i):
        pltpu.async_copy(msg_bufs[buf_i], accum.at[dst_v.at[j]], ssems[buf_i], add=True)
        if with_cnt:
            pltpu.async_copy(ones_v, cnt_accum.at[dst_v.at[j]], ssems[buf_i], add=True)

    def wait_scatter(j, buf_i):
        pltpu.make_async_copy(msg_bufs[buf_i], accum.at[dst_v.at[j]], ssems[buf_i]).wait()
        if with_cnt:
            pltpu.make_async_copy(ones_v, cnt_accum.at[dst_v.at[j]], ssems[buf_i]).wait()

    start_fetch(0, 0)

    def pipe_body(j2, carry):
        j = j2 * 2
        start_fetch(j + 1, 1)
        wait_fetch(j, 0)

        @pl.when(j2 > 0)
        def _():
            wait_scatter(j - 2, 0)
        compute_block(j, 0)
        start_scatter(j, 0)

        @pl.when(j2 < nb2 - 1)
        def _():
            start_fetch(j + 2, 0)
        wait_fetch(j + 1, 1)

        @pl.when(j2 > 0)
        def _():
            wait_scatter(j - 1, 1)
        compute_block(j + 1, 1)
        start_scatter(j + 1, 1)
        return carry
    lax.fori_loop(0, nb2, pipe_body, 0)
    wait_scatter(2 * nb2 - 2, 0)
    wait_scatter(2 * nb2 - 1, 1)

    plsc.subcore_barrier()
    pltpu.sync_copy(accum.at[pl.ds(s * STRIPE, STRIPE)],
                    acc_out.at[c, pl.ds(s * STRIPE, STRIPE)])
    if with_cnt:
        pltpu.sync_copy(cnt_accum.at[pl.ds(s * STRIPE, STRIPE)],
                        cnt_out.at[c, pl.ds(s * STRIPE, STRIPE)])


def _make_sc(with_cnt):
    mesh = plsc.VectorSubcoreMesh(core_axis_name="c", subcore_axis_name="s",
                                  num_cores=2, num_subcores=16)
    out_type = [jax.ShapeDtypeStruct((2, NP, DH), jnp.float32)]
    if with_cnt:
        out_type.append(jax.ShapeDtypeStruct((2, NP, DH), jnp.float32))
    scratch = [
        pltpu.VMEM((NBLK0, BE), jnp.int32),
        pltpu.VMEM((NBLK0, BE), jnp.int32),
        [pltpu.VMEM((K, BE), jnp.float32)] * 2,
        [pltpu.VMEM((BE, YP), jnp.int32)] * 2,
        [pltpu.VMEM((BE, DH), jnp.float32)] * 2,
        pltpu.VMEM((BE, DH), jnp.float32),
        pltpu.VMEM_SHARED((NP, DH), jnp.float32),
    ]
    if with_cnt:
        scratch.append(pltpu.VMEM_SHARED((NP, DH), jnp.float32))
    scratch += [pltpu.SemaphoreType.DMA] * 4
    return pl.kernel(
        functools.partial(_sc_body, with_cnt),
        out_type=out_type,
        mesh=mesh,
        scratch_types=scratch,
        compiler_params=pltpu.CompilerParams(use_tc_tiling_on_sc=False),
    )


@functools.lru_cache(maxsize=None)
def _sc_pass_fn(with_cnt):
    return _make_sc(with_cnt)


def _combine_body(acc_ref, cnt_ref, r_ref, ga_ref, gb_ref, rt_ref, bi_ref,
                  y_ref, rn_ref):
    s = acc_ref[0] + acc_ref[1]
    c = cnt_ref[0] + cnt_ref[1]
    aggr = s / jnp.maximum(c, 1.0)
    h = jax.nn.relu(aggr + r_ref[...])
    ya = jnp.dot(h, ga_ref[...], preferred_element_type=jnp.float32)
    yb = jnp.dot(h, gb_ref[...], preferred_element_type=jnp.float32)
    y_ref[...] = _pack_pair(ya, yb)
    rn_ref[...] = jnp.dot(h, rt_ref[...], preferred_element_type=jnp.float32) + bi_ref[...]


def _combine(acc, cnt, r, ga, gb, root_next, bias_next):
    full = lambda shape: pl.BlockSpec(shape, lambda i: (0,) * len(shape))
    return pl.pallas_call(
        _combine_body,
        grid=(GRID,),
        in_specs=[
            pl.BlockSpec((2, NB, DH), lambda i: (0, i, 0)),
            pl.BlockSpec((2, NB, DH), lambda i: (0, i, 0)),
            pl.BlockSpec((NB, DH), lambda i: (i, 0)),
            full((DH, YP)), full((DH, YP)), full((DH, DH)), full((1, DH)),
        ],
        out_specs=[
            pl.BlockSpec((NB, YP), lambda i: (i, 0)),
            pl.BlockSpec((NB, DH), lambda i: (i, 0)),
        ],
        out_shape=[
            jax.ShapeDtypeStruct((N, YP), jnp.int32),
            jax.ShapeDtypeStruct((N, DH), jnp.float32),
        ],
    )(acc, cnt, r, ga, gb, root_next, bias_next)


def _combine_out_body(acc_ref, cnt_ref, r_ref, w1_ref, b1_ref, w2_ref, b2_ref, o_ref):
    s = acc_ref[0] + acc_ref[1]
    c = cnt_ref[0] + cnt_ref[1]
    aggr = s / jnp.maximum(c, 1.0)
    h = jax.nn.relu(aggr + r_ref[...])
    t = jax.nn.relu(jnp.dot(h, w1_ref[...], preferred_element_type=jnp.float32) + b1_ref[...])
    o_ref[...] = jnp.dot(t, w2_ref[...], preferred_element_type=jnp.float32) + b2_ref[...]


def _combine_out(acc, cnt, r, W_o1, b_o1, W_o2, b_o2):
    full = lambda shape: pl.BlockSpec(shape, lambda i: (0,) * len(shape))
    return pl.pallas_call(
        _combine_out_body,
        grid=(GRID,),
        in_specs=[
            pl.BlockSpec((2, NB, DH), lambda i: (0, i, 0)),
            pl.BlockSpec((2, NB, DH), lambda i: (0, i, 0)),
            pl.BlockSpec((NB, DH), lambda i: (i, 0)),
            full((DH, DH)), full((1, DH)), full((DH, DF)), full((1, DF)),
        ],
        out_specs=pl.BlockSpec((NB, DF), lambda i: (i, 0)),
        out_shape=jax.ShapeDtypeStruct((N, DF), jnp.float32),
    )(acc, cnt, r, W_o1, b_o1, W_o2, b_o2)


def kernel(x, edge_index, edge_attr, W_in1, b_in1, ga_in1, be_in1, W_in2, b_in2,
           ga_in2, be_in2, g0, mu0, sigma0, root0, bias0, g1, mu1, sigma1, root1,
           bias1, g2, mu2, sigma2, root2, bias2, W_o1, b_o1, W_o2, b_o2):
    src = edge_index[0]
    dst = edge_index[1]

    sc = 1.0 / jnp.sqrt(jnp.float32(1.0 + 1e-5))
    W1f = W_in1 * (ga_in1 * sc)[None, :]
    b1f = (b_in1 * ga_in1 * sc + be_in1)[None, :]
    W2f = W_in2 * (ga_in2 * sc)[None, :]
    b2f = (b_in2 * ga_in2 * sc + be_in2)[None, :]

    def pack_consts(mu, sigma):
        a = -0.5 / (1e-15 + sigma * sigma)
        A = jnp.pad(a, ((0, 2), (0, 0)))
        B = jnp.pad(-2.0 * a * mu, ((0, 2), (0, 0)))
        C = jnp.pad((a * mu * mu).sum(-1), (0, 2))
        return jnp.concatenate([A, B, C[:, None], jnp.zeros((8, 7), jnp.float32)], axis=1)

    c0 = pack_consts(mu0, sigma0)
    c1 = pack_consts(mu1, sigma1)
    c2 = pack_consts(mu2, sigma2)
    eaT = jnp.pad(edge_attr.T, ((0, 0), (0, EP - E)))

    def make_table(v, padval):
        blk = jnp.concatenate(
            [v, jnp.full((EP - E,), padval, jnp.int32)]).reshape(NBLKT, BE)
        sc0 = blk[:16 * NBLK0].reshape(16, NBLK0, BE)
        sc1 = blk[16 * NBLK0:].reshape(16, NBLK1, BE)
        sc1 = jnp.pad(sc1, ((0, 0), (0, NBLK0 - NBLK1), (0, 0)),
                      constant_values=padval)
        return jnp.concatenate([sc0, sc1], 0).reshape(32 * NBLK0, BE)

    src_p = make_table(src, 0)
    dst_p = make_table(dst, N)

    gT0 = _gauss(eaT, c0)
    ca, cb = jnp.asarray(_COLS_A), jnp.asarray(_COLS_B)
    y, r = _lin_in(x, W1f, b1f, W2f, b2f, g0[:, ca], g0[:, cb], root0, bias0[None, :])

    acc, cnt = _sc_pass_fn(True)(y, src_p, dst_p, gT0)
    gT1 = _gauss(eaT, c1)
    y, r = _combine(acc, cnt, r, g1[:, ca], g1[:, cb], root1, bias1[None, :])
    (acc,) = _sc_pass_fn(False)(y, src_p, dst_p, gT1)
    gT2 = _gauss(eaT, c2)
    y, r = _combine(acc, cnt, r, g2[:, ca], g2[:, cb], root2, bias2[None, :])
    (acc,) = _sc_pass_fn(False)(y, src_p, dst_p, gT2)
    out = _combine_out(acc, cnt, r, W_o1, b_o1[None, :], W_o2, b_o2[None, :])
    return out


# speedup vs baseline: 7.0754x; 1.1643x over previous
import functools

import jax
import jax.numpy as jnp
import numpy as np
from jax import lax
from jax.experimental import pallas as pl
from jax.experimental.pallas import tpu as pltpu
from jax.experimental.pallas import tpu_sc as plsc

N = 10000
E = 320000
DF = 128
DH = 16
ED = 4
K = 6
YW = K * DH

NB = 400
GRID = N // NB
EB = 10240
NWORK = 32
BE = 128
NBLK0 = 106
NBLK1 = 54
NBLKT = 16 * NBLK0 + 16 * NBLK1
EP = NBLKT * BE
NP = 10240
STRIPE = NP // 16
YP = YW // 2

_COLS_A = np.concatenate([np.arange(16) + (2 * t) * 16 for t in range(K // 2)])
_COLS_B = np.concatenate([np.arange(16) + (2 * t + 1) * 16 for t in range(K // 2)])


def _pack_pair(ya, yb):
    ua = lax.bitcast_convert_type(ya.astype(jnp.bfloat16), jnp.uint16).astype(jnp.int32)
    ub = lax.bitcast_convert_type(yb.astype(jnp.bfloat16), jnp.uint16).astype(jnp.int32)
    return ua | (ub << 16)


def _lin_in_body(x_ref, w1_ref, b1_ref, w2_ref, b2_ref, ga_ref, gb_ref, rt_ref,
                 bi_ref, y_ref, r_ref):
    h = jnp.dot(x_ref[...], w1_ref[...], preferred_element_type=jnp.float32)
    h = jax.nn.relu(h + b1_ref[...])
    h = jnp.dot(h, w2_ref[...], preferred_element_type=jnp.float32)
    h = jax.nn.relu(h + b2_ref[...])
    ya = jnp.dot(h, ga_ref[...], preferred_element_type=jnp.float32)
    yb = jnp.dot(h, gb_ref[...], preferred_element_type=jnp.float32)
    y_ref[...] = _pack_pair(ya, yb)
    r_ref[...] = jnp.dot(h, rt_ref[...], preferred_element_type=jnp.float32) + bi_ref[...]


def _lin_in(x, W1f, b1f, W2f, b2f, ga, gb, root0, bias0):
    full = lambda shape: pl.BlockSpec(shape, lambda i: (0,) * len(shape))
    return pl.pallas_call(
        _lin_in_body,
        grid=(GRID,),
        in_specs=[
            pl.BlockSpec((NB, DF), lambda i: (i, 0)),
            full((DF, DF)), full((1, DF)), full((DF, DH)), full((1, DH)),
            full((DH, YP)), full((DH, YP)), full((DH, DH)), full((1, DH)),
        ],
        out_specs=[
            pl.BlockSpec((NB, YP), lambda i: (i, 0)),
            pl.BlockSpec((NB, DH), lambda i: (i, 0)),
        ],
        out_shape=[
            jax.ShapeDtypeStruct((N, YP), jnp.int32),
            jax.ShapeDtypeStruct((N, DH), jnp.float32),
        ],
    )(x, W1f, b1f, W2f, b2f, ga, gb, root0, bias0)


def _gauss_body(ea_ref, c_ref, o_ref):
    ea = ea_ref[...]
    ea2 = ea * ea
    cc = c_ref[...]
    acc = cc[:, 8:9]
    for d in range(ED):
        acc = acc + cc[:, d:d + 1] * ea2[d:d + 1, :] + cc[:, ED + d:ED + d + 1] * ea[d:d + 1, :]
    o_ref[...] = jnp.exp(acc)


def _gauss(eaT, c):
    full = lambda shape: pl.BlockSpec(shape, lambda i: (0,) * len(shape))
    return pl.pallas_call(
        _gauss_body,
        grid=(EP // EB,),
        in_specs=[pl.BlockSpec((4, EB), lambda i: (0, i)), full((8, 16))],
        out_specs=pl.BlockSpec((8, EB), lambda i: (0, i)),
        out_shape=jax.ShapeDtypeStruct((8, EP), jnp.float32),
    )(eaT, c)


def _sc_body(with_cnt, *refs):
    if with_cnt:
        (y_hbm, src_hbm, dst_hbm, g_hbm, acc_out, cnt_out,
         src_v, dst_v, g_bufs, y_bufs, msg_bufs, fill_v, accum, cnt_accum,
         sem0, sem1, ssem0, ssem1) = refs
    else:
        (y_hbm, src_hbm, dst_hbm, g_hbm, acc_out,
         src_v, dst_v, g_bufs, y_bufs, msg_bufs, fill_v, accum,
         sem0, sem1, ssem0, ssem1) = refs
        cnt_out = cnt_accum = None
    c = lax.axis_index("c")
    s = lax.axis_index("s")
    w = c * 16 + s
    nb2 = jnp.where(c == 0, NBLK0 // 2, NBLK1 // 2)
    gb0 = jnp.where(c == 0, s * NBLK0, 16 * NBLK0 + s * NBLK1)

    pltpu.sync_copy(src_hbm.at[pl.ds(w * NBLK0, NBLK0), :], src_v)
    pltpu.sync_copy(dst_hbm.at[pl.ds(w * NBLK0, NBLK0), :], dst_v)

    def fill_rows(val):
        def body(j, carry):
            fill_v[j, :] = jnp.full((16,), val, jnp.float32)
            return carry
        lax.fori_loop(0, BE, body, 0)

    fill_rows(0.0)
    for si in range(STRIPE // BE):
        pltpu.sync_copy(fill_v, accum.at[pl.ds(s * STRIPE + si * BE, BE)])
        if with_cnt:
            pltpu.sync_copy(fill_v, cnt_accum.at[pl.ds(s * STRIPE + si * BE, BE)])
    if with_cnt:
        fill_rows(1.0)
    ones_v = fill_v
    plsc.subcore_barrier()

    sems = (sem0, sem1)
    ssems = (ssem0, ssem1)

    def g_slice(j):
        return g_hbm.at[pl.ds(0, K), pl.ds((gb0 + j) * BE, BE)]

    def start_fetch(j, buf_i):
        pltpu.async_copy(y_hbm.at[src_v.at[j]], y_bufs[buf_i], sems[buf_i])
        pltpu.async_copy(g_slice(j), g_bufs[buf_i], sems[buf_i])

    def wait_fetch(j, buf_i):
        pltpu.make_async_copy(y_hbm.at[src_v.at[j]], y_bufs[buf_i], sems[buf_i]).wait()
        pltpu.make_async_copy(g_slice(j), g_bufs[buf_i], sems[buf_i]).wait()

    def compute_block(j, buf_i):
        y_blk = y_bufs[buf_i]
        g_blk = g_bufs[buf_i]
        msg_v = msg_bufs[buf_i]

        def group_body(gi, carry2):
            e0 = gi * 16
            gv = [g_blk[k, pl.ds(e0, 16)] for k in range(K)]
            for jj in range(16):
                e = gi * 16 + jj
                acc = None
                for t in range(K // 2):
                    p = y_blk[e, pl.ds(16 * t, 16)]
                    a = lax.bitcast_convert_type(jnp.left_shift(p, 16), jnp.float32)
                    b = lax.bitcast_convert_type(jnp.bitwise_and(p, jnp.int32(-65536)), jnp.float32)
                    term = gv[2 * t][jj] * a + gv[2 * t + 1][jj] * b
                    acc = term if acc is None else acc + term
                msg_v[e, :] = acc
            return carry2
        lax.fori_loop(0, BE // 16, group_body, 0)

    def start_scatter(j, buf_i):
        pltpu.async_copy(msg_bufs[buf_i], accum.at[dst_v.at[j]], ssems[buf_i], add=True)
        if with_cnt:
            pltpu.async_copy(ones_v, cnt_accum.at[dst_v.at[j]], ssems[buf_i], add=True)

    def wait_scatter(j, buf_i):
        pltpu.make_async_copy(msg_bufs[buf_i], accum.at[dst_v.at[j]], ssems[buf_i]).wait()
        if with_cnt:
            pltpu.make_async_copy(ones_v, cnt_accum.at[dst_v.at[j]], ssems[buf_i]).wait()

    start_fetch(0, 0)

    def pipe_body(j2, carry):
        j = j2 * 2
        start_fetch(j + 1, 1)
        wait_fetch(j, 0)

        @pl.when(j2 > 0)
        def _():
            wait_scatter(j - 2, 0)
        compute_block(j, 0)
        start_scatter(j, 0)

        @pl.when(j2 < nb2 - 1)
        def _():
            start_fetch(j + 2, 0)
        wait_fetch(j + 1, 1)

        @pl.when(j2 > 0)
        def _():
            wait_scatter(j - 1, 1)
        compute_block(j + 1, 1)
        start_scatter(j + 1, 1)
        return carry
    lax.fori_loop(0, nb2, pipe_body, 0)
    wait_scatter(2 * nb2 - 2, 0)
    wait_scatter(2 * nb2 - 1, 1)

    plsc.subcore_barrier()
    pltpu.sync_copy(accum.at[pl.ds(s * STRIPE, STRIPE)],
                    acc_out.at[c, pl.ds(s * STRIPE, STRIPE)])
    if with_cnt:
        pltpu.sync_copy(cnt_accum.at[pl.ds(s * STRIPE, STRIPE)],
                        cnt_out.at[c, pl.ds(s * STRIPE, STRIPE)])


def _make_sc(with_cnt):
    mesh = plsc.VectorSubcoreMesh(core_axis_name="c", subcore_axis_name="s",
                                  num_cores=2, num_subcores=16)
    out_type = [jax.ShapeDtypeStruct((2, NP, DH), jnp.float32)]
    if with_cnt:
        out_type.append(jax.ShapeDtypeStruct((2, NP, DH), jnp.float32))
    scratch = [
        pltpu.VMEM((NBLK0, BE), jnp.int32),
        pltpu.VMEM((NBLK0, BE), jnp.int32),
        [pltpu.VMEM((K, BE), jnp.float32)] * 2,
        [pltpu.VMEM((BE, YP), jnp.int32)] * 2,
        [pltpu.VMEM((BE, DH), jnp.float32)] * 2,
        pltpu.VMEM((BE, DH), jnp.float32),
        pltpu.VMEM_SHARED((NP, DH), jnp.float32),
    ]
    if with_cnt:
        scratch.append(pltpu.VMEM_SHARED((NP, DH), jnp.float32))
    scratch += [pltpu.SemaphoreType.DMA] * 4
    return pl.kernel(
        functools.partial(_sc_body, with_cnt),
        out_type=out_type,
        mesh=mesh,
        scratch_types=scratch,
        compiler_params=pltpu.CompilerParams(use_tc_tiling_on_sc=False),
    )


@functools.lru_cache(maxsize=None)
def _sc_pass_fn(with_cnt):
    return _make_sc(with_cnt)


def _combine_body(acc_ref, cnt_ref, r_ref, ga_ref, gb_ref, rt_ref, bi_ref,
                  y_ref, rn_ref):
    s = acc_ref[0] + acc_ref[1]
    c = cnt_ref[0] + cnt_ref[1]
    aggr = s / jnp.maximum(c, 1.0)
    h = jax.nn.relu(aggr + r_ref[...])
    ya = jnp.dot(h, ga_ref[...], preferred_element_type=jnp.float32)
    yb = jnp.dot(h, gb_ref[...], preferred_element_type=jnp.float32)
    y_ref[...] = _pack_pair(ya, yb)
    rn_ref[...] = jnp.dot(h, rt_ref[...], preferred_element_type=jnp.float32) + bi_ref[...]


def _combine(acc, cnt, r, ga, gb, root_next, bias_next):
    full = lambda shape: pl.BlockSpec(shape, lambda i: (0,) * len(shape))
    return pl.pallas_call(
        _combine_body,
        grid=(GRID,),
        in_specs=[
            pl.BlockSpec((2, NB, DH), lambda i: (0, i, 0)),
            pl.BlockSpec((2, NB, DH), lambda i: (0, i, 0)),
            pl.BlockSpec((NB, DH), lambda i: (i, 0)),
            full((DH, YP)), full((DH, YP)), full((DH, DH)), full((1, DH)),
        ],
        out_specs=[
            pl.BlockSpec((NB, YP), lambda i: (i, 0)),
            pl.BlockSpec((NB, DH), lambda i: (i, 0)),
        ],
        out_shape=[
            jax.ShapeDtypeStruct((N, YP), jnp.int32),
            jax.ShapeDtypeStruct((N, DH), jnp.float32),
        ],
    )(acc, cnt, r, ga, gb, root_next, bias_next)


def _combine_out_body(acc_ref, cnt_ref, r_ref, w1_ref, b1_ref, w2_ref, b2_ref, o_ref):
    s = acc_ref[0] + acc_ref[1]
    c = cnt_ref[0] + cnt_ref[1]
    aggr = s / jnp.maximum(c, 1.0)
    h = jax.nn.relu(aggr + r_ref[...])
    t = jax.nn.relu(jnp.dot(h, w1_ref[...], preferred_element_type=jnp.float32) + b1_ref[...])
    o_ref[...] = jnp.dot(t, w2_ref[...], preferred_element_type=jnp.float32) + b2_ref[...]


def _combine_out(acc, cnt, r, W_o1, b_o1, W_o2, b_o2):
    full = lambda shape: pl.BlockSpec(shape, lambda i: (0,) * len(shape))
    return pl.pallas_call(
        _combine_out_body,
        grid=(GRID,),
        in_specs=[
            pl.BlockSpec((2, NB, DH), lambda i: (0, i, 0)),
            pl.BlockSpec((2, NB, DH), lambda i: (0, i, 0)),
            pl.BlockSpec((NB, DH), lambda i: (i, 0)),
            full((DH, DH)), full((1, DH)), full((DH, DF)), full((1, DF)),
        ],
        out_specs=pl.BlockSpec((NB, DF), lambda i: (i, 0)),
        out_shape=jax.ShapeDtypeStruct((N, DF), jnp.float32),
    )(acc, cnt, r, W_o1, b_o1, W_o2, b_o2)


def kernel(x, edge_index, edge_attr, W_in1, b_in1, ga_in1, be_in1, W_in2, b_in2,
           ga_in2, be_in2, g0, mu0, sigma0, root0, bias0, g1, mu1, sigma1, root1,
           bias1, g2, mu2, sigma2, root2, bias2, W_o1, b_o1, W_o2, b_o2):
    src = edge_index[0]
    dst = edge_index[1]

    sc = 1.0 / jnp.sqrt(jnp.float32(1.0 + 1e-5))
    W1f = W_in1 * (ga_in1 * sc)[None, :]
    b1f = (b_in1 * ga_in1 * sc + be_in1)[None, :]
    W2f = W_in2 * (ga_in2 * sc)[None, :]
    b2f = (b_in2 * ga_in2 * sc + be_in2)[None, :]

    def pack_consts(mu, sigma):
        a = -0.5 / (1e-15 + sigma * sigma)
        A = jnp.pad(a, ((0, 2), (0, 0)))
        B = jnp.pad(-2.0 * a * mu, ((0, 2), (0, 0)))
        C = jnp.pad((a * mu * mu).sum(-1), (0, 2))
        return jnp.concatenate([A, B, C[:, None], jnp.zeros((8, 7), jnp.float32)], axis=1)

    c0 = pack_consts(mu0, sigma0)
    c1 = pack_consts(mu1, sigma1)
    c2 = pack_consts(mu2, sigma2)
    eaT = jnp.pad(edge_attr.T, ((0, 0), (0, EP - E)))

    def make_table(v, padval):
        blk = jnp.concatenate(
            [v, jnp.full((EP - E,), padval, jnp.int32)]).reshape(NBLKT, BE)
        sc0 = blk[:16 * NBLK0].reshape(16, NBLK0, BE)
        sc1 = blk[16 * NBLK0:].reshape(16, NBLK1, BE)
        sc1 = jnp.pad(sc1, ((0, 0), (0, NBLK0 - NBLK1), (0, 0)),
                      constant_values=padval)
        return jnp.concatenate([sc0, sc1], 0).reshape(32 * NBLK0, BE)

    src_p = make_table(src, 0)
    dst_p = make_table(dst, N)

    gT0 = _gauss(eaT, c0)
    ca, cb = jnp.asarray(_COLS_A), jnp.asarray(_COLS_B)
    y, r = _lin_in(x, W1f, b1f, W2f, b2f, g0[:, ca], g0[:, cb], root0, bias0[None, :])

    acc, cnt = _sc_pass_fn(True)(y, src_p, dst_p, gT0)
    gT1 = _gauss(eaT, c1)
    y, r = _combine(acc, cnt, r, g1[:, ca], g1[:, cb], root1, bias1[None, :])
    (acc,) = _sc_pass_fn(False)(y, src_p, dst_p, gT1)
    gT2 = _gauss(eaT, c2)
    y, r = _combine(acc, cnt, r, g2[:, ca], g2[:, cb], root2, bias2[None, :])
    (acc,) = _sc_pass_fn(False)(y, src_p, dst_p, gT2)
    out = _combine_out(acc, cnt, r, W_o1, b_o1[None, :], W_o2, b_o2[None, :])
    return out


# speedup vs baseline: 7.5094x; 1.0613x over previous
import functools

import jax
import jax.numpy as jnp
import numpy as np
from jax import lax
from jax.experimental import pallas as pl
from jax.experimental.pallas import tpu as pltpu
from jax.experimental.pallas import tpu_sc as plsc

N = 10000
E = 320000
DF = 128
DH = 16
ED = 4
K = 6
YW = K * DH

NB = 2000
GRID = N // NB
EB = 10240
NWORK = 32
BE = 128
NBLK0 = 106
NBLK1 = 54
NBLKT = 16 * NBLK0 + 16 * NBLK1
EP = NBLKT * BE
NP = 10240
STRIPE = NP // 16
YP = YW // 2

_COLS_A = np.concatenate([np.arange(16) + (2 * t) * 16 for t in range(K // 2)])
_COLS_B = np.concatenate([np.arange(16) + (2 * t + 1) * 16 for t in range(K // 2)])


def _pack_pair(ya, yb):
    ua = lax.bitcast_convert_type(ya.astype(jnp.bfloat16), jnp.uint16).astype(jnp.int32)
    ub = lax.bitcast_convert_type(yb.astype(jnp.bfloat16), jnp.uint16).astype(jnp.int32)
    return ua | (ub << 16)


def _lin_in_body(x_ref, w1_ref, b1_ref, w2_ref, b2_ref, ga_ref, gb_ref, rt_ref,
                 bi_ref, y_ref, r_ref):
    h = jnp.dot(x_ref[...], w1_ref[...], preferred_element_type=jnp.float32)
    h = jax.nn.relu(h + b1_ref[...])
    h = jnp.dot(h, w2_ref[...], preferred_element_type=jnp.float32)
    h = jax.nn.relu(h + b2_ref[...])
    ya = jnp.dot(h, ga_ref[...], preferred_element_type=jnp.float32)
    yb = jnp.dot(h, gb_ref[...], preferred_element_type=jnp.float32)
    y_ref[...] = _pack_pair(ya, yb)
    r_ref[...] = jnp.dot(h, rt_ref[...], preferred_element_type=jnp.float32) + bi_ref[...]


def _lin_in(x, W1f, b1f, W2f, b2f, ga, gb, root0, bias0):
    full = lambda shape: pl.BlockSpec(shape, lambda i: (0,) * len(shape))
    return pl.pallas_call(
        _lin_in_body,
        grid=(GRID,),
        in_specs=[
            pl.BlockSpec((NB, DF), lambda i: (i, 0)),
            full((DF, DF)), full((1, DF)), full((DF, DH)), full((1, DH)),
            full((DH, YP)), full((DH, YP)), full((DH, DH)), full((1, DH)),
        ],
        out_specs=[
            pl.BlockSpec((NB, YP), lambda i: (i, 0)),
            pl.BlockSpec((NB, DH), lambda i: (i, 0)),
        ],
        out_shape=[
            jax.ShapeDtypeStruct((N, YP), jnp.int32),
            jax.ShapeDtypeStruct((N, DH), jnp.float32),
        ],
    )(x, W1f, b1f, W2f, b2f, ga, gb, root0, bias0)


def _gauss_body(ea_ref, c_ref, o_ref):
    ea = ea_ref[...]
    ea2 = ea * ea
    cc = c_ref[...]
    acc = cc[:, 8:9]
    for d in range(ED):
        acc = acc + cc[:, d:d + 1] * ea2[d:d + 1, :] + cc[:, ED + d:ED + d + 1] * ea[d:d + 1, :]
    o_ref[...] = jnp.exp(acc)


def _gauss(eaT, c):
    full = lambda shape: pl.BlockSpec(shape, lambda i: (0,) * len(shape))
    return pl.pallas_call(
        _gauss_body,
        grid=(EP // EB,),
        in_specs=[pl.BlockSpec((4, EB), lambda i: (0, i)), full((8, 16))],
        out_specs=pl.BlockSpec((8, EB), lambda i: (0, i)),
        out_shape=jax.ShapeDtypeStruct((8, EP), jnp.float32),
    )(eaT, c)


def _sc_body(with_cnt, *refs):
    if with_cnt:
        (y_hbm, src_hbm, dst_hbm, g_hbm, acc_out, cnt_out,
         src_v, dst_v, g_bufs, y_bufs, msg_bufs, fill_v, accum, cnt_accum,
         sem0, sem1, ssem0, ssem1) = refs
    else:
        (y_hbm, src_hbm, dst_hbm, g_hbm, acc_out,
         src_v, dst_v, g_bufs, y_bufs, msg_bufs, fill_v, accum,
         sem0, sem1, ssem0, ssem1) = refs
        cnt_out = cnt_accum = None
    c = lax.axis_index("c")
    s = lax.axis_index("s")
    w = c * 16 + s
    nb2 = jnp.where(c == 0, NBLK0 // 2, NBLK1 // 2)
    gb0 = jnp.where(c == 0, s * NBLK0, 16 * NBLK0 + s * NBLK1)

    pltpu.sync_copy(src_hbm.at[pl.ds(w * NBLK0, NBLK0), :], src_v)
    pltpu.sync_copy(dst_hbm.at[pl.ds(w * NBLK0, NBLK0), :], dst_v)

    def fill_rows(val):
        def body(j, carry):
            fill_v[j, :] = jnp.full((16,), val, jnp.float32)
            return carry
        lax.fori_loop(0, BE, body, 0)

    fill_rows(0.0)
    for si in range(STRIPE // BE):
        pltpu.sync_copy(fill_v, accum.at[pl.ds(s * STRIPE + si * BE, BE)])
        if with_cnt:
            pltpu.sync_copy(fill_v, cnt_accum.at[pl.ds(s * STRIPE + si * BE, BE)])
    if with_cnt:
        fill_rows(1.0)
    ones_v = fill_v
    plsc.subcore_barrier()

    sems = (sem0, sem1)
    ssems = (ssem0, ssem1)

    def g_slice(j):
        return g_hbm.at[pl.ds(0, K), pl.ds((gb0 + j) * BE, BE)]

    def start_fetch(j, buf_i):
        pltpu.async_copy(y_hbm.at[src_v.at[j]], y_bufs[buf_i], sems[buf_i])
        pltpu.async_copy(g_slice(j), g_bufs[buf_i], sems[buf_i])

    def wait_fetch(j, buf_i):
        pltpu.make_async_copy(y_hbm.at[src_v.at[j]], y_bufs[buf_i], sems[buf_i]).wait()
        pltpu.make_async_copy(g_slice(j), g_bufs[buf_i], sems[buf_i]).wait()

    def compute_block(j, buf_i):
        y_blk = y_bufs[buf_i]
        g_blk = g_bufs[buf_i]
        msg_v = msg_bufs[buf_i]

        def group_body(gi, carry2):
            e0 = gi * 16
            gv = [g_blk[k, pl.ds(e0, 16)] for k in range(K)]
            for jj in range(16):
                e = gi * 16 + jj
                acc = None
                for t in range(K // 2):
                    p = y_blk[e, pl.ds(16 * t, 16)]
                    a = lax.bitcast_convert_type(jnp.left_shift(p, 16), jnp.float32)
                    b = lax.bitcast_convert_type(jnp.bitwise_and(p, jnp.int32(-65536)), jnp.float32)
                    term = gv[2 * t][jj] * a + gv[2 * t + 1][jj] * b
                    acc = term if acc is None else acc + term
                msg_v[e, :] = acc
            return carry2
        lax.fori_loop(0, BE // 16, group_body, 0)

    def start_scatter(j, buf_i):
        pltpu.async_copy(msg_bufs[buf_i], accum.at[dst_v.at[j]], ssems[buf_i], add=True)
        if with_cnt:
            pltpu.async_copy(ones_v, cnt_accum.at[dst_v.at[j]], ssems[buf_i], add=True)

    def wait_scatter(j, buf_i):
        pltpu.make_async_copy(msg_bufs[buf_i], accum.at[dst_v.at[j]], ssems[buf_i]).wait()
        if with_cnt:
            pltpu.make_async_copy(ones_v, cnt_accum.at[dst_v.at[j]], ssems[buf_i]).wait()

    start_fetch(0, 0)

    def pipe_body(j2, carry):
        j = j2 * 2
        start_fetch(j + 1, 1)
        wait_fetch(j, 0)

        @pl.when(j2 > 0)
        def _():
            wait_scatter(j - 2, 0)
        compute_block(j, 0)
        start_scatter(j, 0)

        @pl.when(j2 < nb2 - 1)
        def _():
            start_fetch(j + 2, 0)
        wait_fetch(j + 1, 1)

        @pl.when(j2 > 0)
        def _():
            wait_scatter(j - 1, 1)
        compute_block(j + 1, 1)
        start_scatter(j + 1, 1)
        return carry
    lax.fori_loop(0, nb2, pipe_body, 0)
    wait_scatter(2 * nb2 - 2, 0)
    wait_scatter(2 * nb2 - 1, 1)

    plsc.subcore_barrier()
    pltpu.sync_copy(accum.at[pl.ds(s * STRIPE, STRIPE)],
                    acc_out.at[c, pl.ds(s * STRIPE, STRIPE)])
    if with_cnt:
        pltpu.sync_copy(cnt_accum.at[pl.ds(s * STRIPE, STRIPE)],
                        cnt_out.at[c, pl.ds(s * STRIPE, STRIPE)])


def _make_sc(with_cnt):
    mesh = plsc.VectorSubcoreMesh(core_axis_name="c", subcore_axis_name="s",
                                  num_cores=2, num_subcores=16)
    out_type = [jax.ShapeDtypeStruct((2, NP, DH), jnp.float32)]
    if with_cnt:
        out_type.append(jax.ShapeDtypeStruct((2, NP, DH), jnp.float32))
    scratch = [
        pltpu.VMEM((NBLK0, BE), jnp.int32),
        pltpu.VMEM((NBLK0, BE), jnp.int32),
        [pltpu.VMEM((K, BE), jnp.float32)] * 2,
        [pltpu.VMEM((BE, YP), jnp.int32)] * 2,
        [pltpu.VMEM((BE, DH), jnp.float32)] * 2,
        pltpu.VMEM((BE, DH), jnp.float32),
        pltpu.VMEM_SHARED((NP, DH), jnp.float32),
    ]
    if with_cnt:
        scratch.append(pltpu.VMEM_SHARED((NP, DH), jnp.float32))
    scratch += [pltpu.SemaphoreType.DMA] * 4
    return pl.kernel(
        functools.partial(_sc_body, with_cnt),
        out_type=out_type,
        mesh=mesh,
        scratch_types=scratch,
        compiler_params=pltpu.CompilerParams(use_tc_tiling_on_sc=False),
    )


@functools.lru_cache(maxsize=None)
def _sc_pass_fn(with_cnt):
    return _make_sc(with_cnt)


def _combine_body(acc_ref, cnt_ref, r_ref, ga_ref, gb_ref, rt_ref, bi_ref,
                  y_ref, rn_ref):
    s = acc_ref[0] + acc_ref[1]
    c = cnt_ref[0] + cnt_ref[1]
    aggr = s / jnp.maximum(c, 1.0)
    h = jax.nn.relu(aggr + r_ref[...])
    ya = jnp.dot(h, ga_ref[...], preferred_element_type=jnp.float32)
    yb = jnp.dot(h, gb_ref[...], preferred_element_type=jnp.float32)
    y_ref[...] = _pack_pair(ya, yb)
    rn_ref[...] = jnp.dot(h, rt_ref[...], preferred_element_type=jnp.float32) + bi_ref[...]


def _combine(acc, cnt, r, ga, gb, root_next, bias_next):
    full = lambda shape: pl.BlockSpec(shape, lambda i: (0,) * len(shape))
    return pl.pallas_call(
        _combine_body,
        grid=(GRID,),
        in_specs=[
            pl.BlockSpec((2, NB, DH), lambda i: (0, i, 0)),
            pl.BlockSpec((2, NB, DH), lambda i: (0, i, 0)),
            pl.BlockSpec((NB, DH), lambda i: (i, 0)),
            full((DH, YP)), full((DH, YP)), full((DH, DH)), full((1, DH)),
        ],
        out_specs=[
            pl.BlockSpec((NB, YP), lambda i: (i, 0)),
            pl.BlockSpec((NB, DH), lambda i: (i, 0)),
        ],
        out_shape=[
            jax.ShapeDtypeStruct((N, YP), jnp.int32),
            jax.ShapeDtypeStruct((N, DH), jnp.float32),
        ],
    )(acc, cnt, r, ga, gb, root_next, bias_next)


def _combine_out_body(acc_ref, cnt_ref, r_ref, w1_ref, b1_ref, w2_ref, b2_ref, o_ref):
    s = acc_ref[0] + acc_ref[1]
    c = cnt_ref[0] + cnt_ref[1]
    aggr = s / jnp.maximum(c, 1.0)
    h = jax.nn.relu(aggr + r_ref[...])
    t = jax.nn.relu(jnp.dot(h, w1_ref[...], preferred_element_type=jnp.float32) + b1_ref[...])
    o_ref[...] = jnp.dot(t, w2_ref[...], preferred_element_type=jnp.float32) + b2_ref[...]


def _combine_out(acc, cnt, r, W_o1, b_o1, W_o2, b_o2):
    full = lambda shape: pl.BlockSpec(shape, lambda i: (0,) * len(shape))
    return pl.pallas_call(
        _combine_out_body,
        grid=(GRID,),
        in_specs=[
            pl.BlockSpec((2, NB, DH), lambda i: (0, i, 0)),
            pl.BlockSpec((2, NB, DH), lambda i: (0, i, 0)),
            pl.BlockSpec((NB, DH), lambda i: (i, 0)),
            full((DH, DH)), full((1, DH)), full((DH, DF)), full((1, DF)),
        ],
        out_specs=pl.BlockSpec((NB, DF), lambda i: (i, 0)),
        out_shape=jax.ShapeDtypeStruct((N, DF), jnp.float32),
    )(acc, cnt, r, W_o1, b_o1, W_o2, b_o2)


def kernel(x, edge_index, edge_attr, W_in1, b_in1, ga_in1, be_in1, W_in2, b_in2,
           ga_in2, be_in2, g0, mu0, sigma0, root0, bias0, g1, mu1, sigma1, root1,
           bias1, g2, mu2, sigma2, root2, bias2, W_o1, b_o1, W_o2, b_o2):
    src = edge_index[0]
    dst = edge_index[1]

    sc = 1.0 / jnp.sqrt(jnp.float32(1.0 + 1e-5))
    W1f = W_in1 * (ga_in1 * sc)[None, :]
    b1f = (b_in1 * ga_in1 * sc + be_in1)[None, :]
    W2f = W_in2 * (ga_in2 * sc)[None, :]
    b2f = (b_in2 * ga_in2 * sc + be_in2)[None, :]

    def pack_consts(mu, sigma):
        a = -0.5 / (1e-15 + sigma * sigma)
        A = jnp.pad(a, ((0, 2), (0, 0)))
        B = jnp.pad(-2.0 * a * mu, ((0, 2), (0, 0)))
        C = jnp.pad((a * mu * mu).sum(-1), (0, 2))
        return jnp.concatenate([A, B, C[:, None], jnp.zeros((8, 7), jnp.float32)], axis=1)

    c0 = pack_consts(mu0, sigma0)
    c1 = pack_consts(mu1, sigma1)
    c2 = pack_consts(mu2, sigma2)
    eaT = jnp.pad(edge_attr.T, ((0, 0), (0, EP - E)))

    def make_table(v, padval):
        blk = jnp.concatenate(
            [v, jnp.full((EP - E,), padval, jnp.int32)]).reshape(NBLKT, BE)
        sc0 = blk[:16 * NBLK0].reshape(16, NBLK0, BE)
        sc1 = blk[16 * NBLK0:].reshape(16, NBLK1, BE)
        sc1 = jnp.pad(sc1, ((0, 0), (0, NBLK0 - NBLK1), (0, 0)),
                      constant_values=padval)
        return jnp.concatenate([sc0, sc1], 0).reshape(32 * NBLK0, BE)

    src_p = make_table(src, 0)
    dst_p = make_table(dst, N)

    gT0 = _gauss(eaT, c0)
    ca, cb = jnp.asarray(_COLS_A), jnp.asarray(_COLS_B)
    y, r = _lin_in(x, W1f, b1f, W2f, b2f, g0[:, ca], g0[:, cb], root0, bias0[None, :])

    acc, cnt = _sc_pass_fn(True)(y, src_p, dst_p, gT0)
    gT1 = _gauss(eaT, c1)
    y, r = _combine(acc, cnt, r, g1[:, ca], g1[:, cb], root1, bias1[None, :])
    (acc,) = _sc_pass_fn(False)(y, src_p, dst_p, gT1)
    gT2 = _gauss(eaT, c2)
    y, r = _combine(acc, cnt, r, g2[:, ca], g2[:, cb], root2, bias2[None, :])
    (acc,) = _sc_pass_fn(False)(y, src_p, dst_p, gT2)
    out = _combine_out(acc, cnt, r, W_o1, b_o1[None, :], W_o2, b_o2[None, :])
    return out
